# Initial kernel scaffold; baseline (speedup 1.0000x reference)
import jax
import jax.numpy as jnp
from jax.experimental import pallas as pl


def kernel(x, pos, edge_attr, edge_index, batch, params):
    raise NotImplementedError("write your pallas kernel here")


# speedup vs baseline: 3.2712x; 3.2712x over previous
import functools

import jax
import jax.numpy as jnp
from jax import lax
from jax.experimental import pallas as pl
from jax.experimental.pallas import tpu as pltpu
from jax.experimental.pallas import tpu_sc as plsc

F32 = jnp.float32
ROW0 = 80
H = 64


def _silu(v):
    return v * (1.0 / (1.0 + jnp.exp(-v)))


def _sc_gather_sum(ti, tj, src, dst, row):
    e = src.shape[0]
    nblk = e // 128
    mesh = plsc.VectorSubcoreMesh(core_axis_name="c", subcore_axis_name="s")

    @functools.partial(
        pl.kernel,
        out_type=jax.ShapeDtypeStruct((e, row), F32),
        mesh=mesh,
        scratch_types=[
            pltpu.VMEM((128,), jnp.int32),
            pltpu.VMEM((128,), jnp.int32),
            pltpu.VMEM((128, row), F32),
            pltpu.VMEM((128, row), F32),
            pltpu.SemaphoreType.DMA,
            pltpu.SemaphoreType.DMA,
        ],
        compiler_params=pltpu.CompilerParams(use_tc_tiling_on_sc=False),
    )
    def k(ti_hbm, tj_hbm, src_hbm, dst_hbm, out_hbm, idx_i, idx_j, ri, rj,
          sem_i, sem_j):
        c = lax.axis_index("c")
        s = lax.axis_index("s")
        wid = s * 2 + c
        nb = (nblk - wid + 31) // 32

        def body(i, carry):
            e0 = (wid + i * 32) * 128
            pltpu.sync_copy(dst_hbm.at[pl.ds(e0, 128)], idx_i)
            pltpu.sync_copy(src_hbm.at[pl.ds(e0, 128)], idx_j)
            cp_i = pltpu.async_copy(ti_hbm.at[idx_i], ri, sem_i)
            cp_j = pltpu.async_copy(tj_hbm.at[idx_j], rj, sem_j)
            cp_i.wait()
            cp_j.wait()

            def add_row(r, carry2):
                for jj in range(row // 16):
                    sl = pl.ds(jj * 16, 16)
                    ri[r, sl] = ri[r, sl] + rj[r, sl]
                return carry2

            lax.fori_loop(0, 128, add_row, 0, unroll=4)
            pltpu.sync_copy(ri, out_hbm.at[pl.ds(e0, 128)])
            return carry

        lax.fori_loop(0, nb, body, 0)

    return k(ti, tj, src, dst)


def _sc_scatter_add(u, dst, zrows, n, row):
    e = u.shape[0]
    nblk = e // 128
    half = nblk // 2
    per = n // 16
    zr = zrows.shape[0]
    nchunk = per // zr
    mesh = plsc.VectorSubcoreMesh(core_axis_name="c", subcore_axis_name="s")

    @functools.partial(
        pl.kernel,
        out_type=jax.ShapeDtypeStruct((2, n, row), F32),
        mesh=mesh,
        scratch_types=[
            pltpu.VMEM((128,), jnp.int32),
            pltpu.VMEM((128, row), F32),
            pltpu.VMEM((zr, row), F32),
            pltpu.VMEM_SHARED((n, row), F32),
        ],
        compiler_params=pltpu.CompilerParams(use_tc_tiling_on_sc=False),
    )
    def k(u_hbm, dst_hbm, z_hbm, out_hbm, idx_v, u_v, z_v, acc):
        c = lax.axis_index("c")
        s = lax.axis_index("s")
        pltpu.sync_copy(z_hbm, z_v)
        for kk in range(nchunk):
            pltpu.sync_copy(z_v, acc.at[pl.ds((s * nchunk + kk) * zr, zr)])
        plsc.subcore_barrier()

        nb = (half - s + 15) // 16

        def body(i, carry):
            e0 = (c * half + s + i * 16) * 128
            pltpu.sync_copy(dst_hbm.at[pl.ds(e0, 128)], idx_v)
            pltpu.sync_copy(u_hbm.at[pl.ds(e0, 128)], u_v)
            pltpu.sync_copy(u_v, acc.at[idx_v], add=True)
            return carry

        lax.fori_loop(0, nb, body, 0)
        plsc.subcore_barrier()

        for kk in range(nchunk):
            r0 = (s * nchunk + kk) * zr
            pltpu.sync_copy(acc.at[pl.ds(r0, zr)], z_v)
            pltpu.sync_copy(z_v, out_hbm.at[c, pl.ds(r0, zr)])

    return k(u, dst, zrows)


def _tc_prep_tables(x, pos, wi, wj, bn):
    n, f = x.shape

    def body(x_ref, pos_ref, wi_ref, wj_ref, ti_ref, tj_ref):
        xb = x_ref[...]
        p = pos_ref[...]
        pad = jnp.zeros((bn, ROW0 - H - 3), F32)
        ti_ref[...] = jnp.concatenate(
            [jnp.dot(xb, wi_ref[...], preferred_element_type=F32), p, pad], 1)
        tj_ref[...] = jnp.concatenate(
            [jnp.dot(xb, wj_ref[...], preferred_element_type=F32), -p, pad], 1)

    grid = (n // bn,)
    return pl.pallas_call(
        body,
        grid=grid,
        in_specs=[
            pl.BlockSpec((bn, f), lambda i: (i, 0)),
            pl.BlockSpec((bn, 3), lambda i: (i, 0)),
            pl.BlockSpec((f, H), lambda i: (0, 0)),
            pl.BlockSpec((f, H), lambda i: (0, 0)),
        ],
        out_specs=[
            pl.BlockSpec((bn, ROW0), lambda i: (i, 0)),
            pl.BlockSpec((bn, ROW0), lambda i: (i, 0)),
        ],
        out_shape=[
            jax.ShapeDtypeStruct((n, ROW0), F32),
            jax.ShapeDtypeStruct((n, ROW0), F32),
        ],
    )(x, pos, wi, wj)


def _tc_edge_proj(ea, we0, we1, be):
    e, d = ea.shape

    def body(ea_ref, w0_ref, w1_ref, a0_ref, a1_ref):
        eb = ea_ref[...]
        a0_ref[...] = jnp.dot(eb, w0_ref[...], preferred_element_type=F32)
        a1_ref[...] = jnp.dot(eb, w1_ref[...], preferred_element_type=F32)

    return pl.pallas_call(
        body,
        grid=(e // be,),
        in_specs=[
            pl.BlockSpec((be, d), lambda i: (i, 0)),
            pl.BlockSpec((d, H), lambda i: (0, 0)),
            pl.BlockSpec((d, H), lambda i: (0, 0)),
        ],
        out_specs=[
            pl.BlockSpec((be, H), lambda i: (i, 0)),
            pl.BlockSpec((be, H), lambda i: (i, 0)),
        ],
        out_shape=[
            jax.ShapeDtypeStruct((e, H), F32),
            jax.ShapeDtypeStruct((e, H), F32),
        ],
    )(ea, we0, we1)


def _tc_edge_mlp(sarr, a, emw, emb, cmw, cmb, wr, be, with_coord):
    e = sarr.shape[0]
    orow = ROW0 if with_coord else H

    def body(s_ref, a_ref, emw_ref, emb_ref, cmw_ref, cmb_ref, wr_ref, u_ref):
        sb = s_ref[...]
        xpart = sb[:, :H]
        diff = sb[:, H:H + 3]
        radial = jnp.sum(diff * diff, axis=1, keepdims=True)
        pre = xpart + a_ref[...] + radial * wr_ref[...]
        er = _silu(pre)
        eij = _silu(jnp.dot(er, emw_ref[...], preferred_element_type=F32)
                    + emb_ref[...])
        if with_coord:
            sc = _silu(jnp.sum(eij * cmw_ref[...], axis=1, keepdims=True)
                       + cmb_ref[...])
            u_ref[...] = jnp.concatenate(
                [eij, diff * sc, jnp.ones((be, 1), F32),
                 jnp.zeros((be, ROW0 - H - 4), F32)], 1)
        else:
            u_ref[...] = eij

    return pl.pallas_call(
        body,
        grid=(e // be,),
        in_specs=[
            pl.BlockSpec((be, ROW0), lambda i: (i, 0)),
            pl.BlockSpec((be, H), lambda i: (i, 0)),
            pl.BlockSpec((H, H), lambda i: (0, 0)),
            pl.BlockSpec((1, H), lambda i: (0, 0)),
            pl.BlockSpec((1, H), lambda i: (0, 0)),
            pl.BlockSpec((1, 1), lambda i: (0, 0)),
            pl.BlockSpec((1, H), lambda i: (0, 0)),
        ],
        out_specs=pl.BlockSpec((be, orow), lambda i: (i, 0)),
        out_shape=jax.ShapeDtypeStruct((e, orow), F32),
    )(sarr, a, emw, emb, cmw, cmb, wr)


def _tc_node0(x, pos, acc0, acc1, w1x, w1e, b1, w2, b2, wi1, wj1, bn):
    n, f = x.shape

    def body(x_ref, pos_ref, a0_ref, a1_ref, w1x_ref, w1e_ref, b1_ref,
             w2_ref, b2_ref, wi_ref, wj_ref, x1_ref, ti_ref, tj_ref):
        a0 = a0_ref[...]
        a1 = a1_ref[...]
        e_agg = a0[:, :H] + a1[:, :H]
        csum = a0[:, H:H + 3] + a1[:, H:H + 3]
        cnt = a0[:, H + 3:H + 4] + a1[:, H + 3:H + 4]
        posn = pos_ref[...] + csum / jnp.maximum(cnt, 1.0)
        h = _silu(jnp.dot(x_ref[...], w1x_ref[...], preferred_element_type=F32)
                  + jnp.dot(e_agg, w1e_ref[...], preferred_element_type=F32)
                  + b1_ref[...])
        x1 = jnp.dot(h, w2_ref[...], preferred_element_type=F32) + b2_ref[...]
        x1_ref[...] = x1
        pad = jnp.zeros((bn, ROW0 - H - 3), F32)
        ti_ref[...] = jnp.concatenate(
            [jnp.dot(x1, wi_ref[...], preferred_element_type=F32), posn, pad], 1)
        tj_ref[...] = jnp.concatenate(
            [jnp.dot(x1, wj_ref[...], preferred_element_type=F32), -posn, pad], 1)

    c0 = pl.BlockSpec((H, H), lambda i: (0, 0))
    cb = pl.BlockSpec((1, H), lambda i: (0, 0))
    return pl.pallas_call(
        body,
        grid=(n // bn,),
        in_specs=[
            pl.BlockSpec((bn, f), lambda i: (i, 0)),
            pl.BlockSpec((bn, 3), lambda i: (i, 0)),
            pl.BlockSpec((bn, ROW0), lambda i: (i, 0)),
            pl.BlockSpec((bn, ROW0), lambda i: (i, 0)),
            pl.BlockSpec((f, H), lambda i: (0, 0)),
            c0, cb, c0, cb, c0, c0,
        ],
        out_specs=[
            pl.BlockSpec((bn, H), lambda i: (i, 0)),
            pl.BlockSpec((bn, ROW0), lambda i: (i, 0)),
            pl.BlockSpec((bn, ROW0), lambda i: (i, 0)),
        ],
        out_shape=[
            jax.ShapeDtypeStruct((n, H), F32),
            jax.ShapeDtypeStruct((n, ROW0), F32),
            jax.ShapeDtypeStruct((n, ROW0), F32),
        ],
    )(x, pos, acc0, acc1, w1x, w1e, b1, w2, b2, wi1, wj1)


def _tc_node1_pool(x1, acc0, acc1, batch_col, w1x, w1e, b1, w2, b2, bn):
    n = x1.shape[0]
    g = 64

    def body(x_ref, a0_ref, a1_ref, b_ref, w1x_ref, w1e_ref, b1_ref,
             w2_ref, b2_ref, out_ref):
        e_agg = a0_ref[...] + a1_ref[...]
        h = _silu(jnp.dot(x_ref[...], w1x_ref[...], preferred_element_type=F32)
                  + jnp.dot(e_agg, w1e_ref[...], preferred_element_type=F32)
                  + b1_ref[...])
        x2 = jnp.dot(h, w2_ref[...], preferred_element_type=F32) + b2_ref[...]
        gid = lax.broadcasted_iota(jnp.int32, (1, g), 1)
        oh = (b_ref[...] == gid).astype(F32)
        feat = jnp.concatenate(
            [x2, jnp.ones((bn, 1), F32), jnp.zeros((bn, 7), F32)], 1)
        contrib = lax.dot_general(oh, feat, (((0,), (0,)), ((), ())),
                                  preferred_element_type=F32)

        @pl.when(pl.program_id(0) == 0)
        def _():
            out_ref[...] = contrib

        @pl.when(pl.program_id(0) != 0)
        def _():
            out_ref[...] = out_ref[...] + contrib

    c0 = pl.BlockSpec((H, H), lambda i: (0, 0))
    cb = pl.BlockSpec((1, H), lambda i: (0, 0))
    return pl.pallas_call(
        body,
        grid=(n // bn,),
        in_specs=[
            pl.BlockSpec((bn, H), lambda i: (i, 0)),
            pl.BlockSpec((bn, H), lambda i: (i, 0)),
            pl.BlockSpec((bn, H), lambda i: (i, 0)),
            pl.BlockSpec((bn, 1), lambda i: (i, 0)),
            c0, c0, cb, c0, cb,
        ],
        out_specs=pl.BlockSpec((g, H + 8), lambda i: (0, 0)),
        out_shape=jax.ShapeDtypeStruct((g, H + 8), F32),
    )(x1, acc0, acc1, batch_col, w1x, w1e, b1, w2, b2)


def _tc_final(pooled, w1, b1, w2, b2):
    g = pooled.shape[0]
    out_c = w2.shape[1]

    def body(p_ref, w1_ref, b1_ref, w2_ref, b2_ref, o_ref):
        p = p_ref[...]
        xg = p[:, :H] / jnp.maximum(p[:, H:H + 1], 1.0)
        m = jnp.max(xg, axis=1, keepdims=True)
        z = xg - m
        xl = z - jnp.log(jnp.sum(jnp.exp(z), axis=1, keepdims=True))
        hh = jnp.maximum(
            jnp.dot(xl, w1_ref[...], preferred_element_type=F32) + b1_ref[...],
            0.0)
        o_ref[...] = (jnp.dot(hh, w2_ref[...], preferred_element_type=F32)
                      + b2_ref[...])

    return pl.pallas_call(
        body,
        out_shape=jax.ShapeDtypeStruct((g, out_c), F32),
    )(pooled, w1, b1, w2, b2)


def kernel(x, pos, edge_attr, edge_index, batch, params):
    n, f = x.shape
    e = edge_index.shape[1]
    d = edge_attr.shape[1]
    src = edge_index[0]
    dst = edge_index[1]

    p0, p1, po = params['l0'], params['l1'], params['out']
    w0 = p0['mlp_W']
    wi0, wj0 = w0[:f], w0[f:2 * f]
    we0, wr0 = w0[2 * f:2 * f + d], w0[2 * f + d].reshape(1, H)
    w1 = p1['mlp_W']
    wi1, wj1 = w1[:H], w1[H:2 * H]
    we1, wr1 = w1[2 * H:2 * H + d], w1[2 * H + d].reshape(1, H)

    zeros80 = jnp.zeros((125, ROW0), F32)
    zeros64 = jnp.zeros((125, H), F32)
    batch_col = batch.reshape(n, 1)

    ti0, tj0 = _tc_prep_tables(x, pos, wi0, wj0, bn=2000)
    a0, a1 = _tc_edge_proj(edge_attr, we0, we1, be=2000)

    s0 = _sc_gather_sum(ti0, tj0, src, dst, ROW0)
    u0 = _tc_edge_mlp(s0, a0, p0['em_W'], p0['em_b'].reshape(1, H),
                      p0['cm_W'].reshape(1, H), p0['cm_b'].reshape(1, 1),
                      wr0, be=2000, with_coord=True)
    acc = _sc_scatter_add(u0, dst, zeros80, n, ROW0)
    x1, ti1, tj1 = _tc_node0(
        x, pos, acc[0], acc[1],
        p0['nm_W1'][:f], p0['nm_W1'][f:], p0['nm_b1'].reshape(1, H),
        p0['nm_W2'], p0['nm_b2'].reshape(1, H), wi1, wj1, bn=2000)

    s1 = _sc_gather_sum(ti1, tj1, src, dst, ROW0)
    u1 = _tc_edge_mlp(s1, a1, p1['em_W'], p1['em_b'].reshape(1, H),
                      p1['cm_W'].reshape(1, H), p1['cm_b'].reshape(1, 1),
                      wr1, be=2000, with_coord=False)
    acc1 = _sc_scatter_add(u1, dst, zeros64, n, H)
    pooled = _tc_node1_pool(
        x1, acc1[0], acc1[1], batch_col,
        p1['nm_W1'][:H], p1['nm_W1'][H:], p1['nm_b1'].reshape(1, H),
        p1['nm_W2'], p1['nm_b2'].reshape(1, H), bn=2000)

    return _tc_final(pooled, po['W1'], po['b1'].reshape(1, H),
                     po['W2'], po['b2'].reshape(1, po['W2'].shape[1]))


# speedup vs baseline: 4.0918x; 1.2509x over previous
import functools

import jax
import jax.numpy as jnp
from jax import lax
from jax.experimental import pallas as pl
from jax.experimental.pallas import tpu as pltpu
from jax.experimental.pallas import tpu_sc as plsc

F32 = jnp.float32
ROW = 128
H = 64


def _silu(v):
    return v * (1.0 / (1.0 + jnp.exp(-v)))


def _sc_gather_sum(ti, tj, src, dst):
    e = src.shape[0]
    nblk = e // 128
    maxp = (nblk // 32 + 2) // 2
    mesh = plsc.VectorSubcoreMesh(core_axis_name="c", subcore_axis_name="s")

    @functools.partial(
        pl.kernel,
        out_type=jax.ShapeDtypeStruct((e, ROW), F32),
        mesh=mesh,
        scratch_types=[
            pltpu.VMEM((128,), jnp.int32), pltpu.VMEM((128,), jnp.int32),
            pltpu.VMEM((128,), jnp.int32), pltpu.VMEM((128,), jnp.int32),
            pltpu.VMEM((128, ROW), F32), pltpu.VMEM((128, ROW), F32),
            pltpu.VMEM((128, ROW), F32), pltpu.VMEM((128, ROW), F32),
            pltpu.SemaphoreType.DMA, pltpu.SemaphoreType.DMA,
            pltpu.SemaphoreType.DMA, pltpu.SemaphoreType.DMA,
            pltpu.SemaphoreType.DMA, pltpu.SemaphoreType.DMA,
        ],
    )
    def k(ti_hbm, tj_hbm, src_hbm, dst_hbm, out_hbm,
          id0, is0, id1, is1, ri0, rj0, ri1, rj1,
          gi0, gj0, gi1, gj1, ws0, ws1):
        c = lax.axis_index("c")
        s = lax.axis_index("s")
        wid = s * 2 + c
        nb = (nblk - wid + 31) // 32

        def start(it, idx_d, idx_s, ri, rj, gi, gj, ws):
            @pl.when(jnp.logical_and(it < nb, it >= 2))
            def _():
                pltpu.make_async_copy(ri, out_hbm.at[pl.ds(0, 128)], ws).wait()

            @pl.when(it < nb)
            def _():
                e0 = (wid + it * 32) * 128
                pltpu.sync_copy(dst_hbm.at[pl.ds(e0, 128)], idx_d)
                pltpu.sync_copy(src_hbm.at[pl.ds(e0, 128)], idx_s)
                pltpu.make_async_copy(ti_hbm.at[idx_d], ri, gi).start()
                pltpu.make_async_copy(tj_hbm.at[idx_s], rj, gj).start()

        def finish(it, ri, rj, gi, gj, ws):
            @pl.when(it < nb)
            def _():
                e0 = (wid + it * 32) * 128
                pltpu.make_async_copy(ti_hbm.at[pl.ds(0, 128)], ri, gi).wait()
                pltpu.make_async_copy(tj_hbm.at[pl.ds(0, 128)], rj, gj).wait()

                def row(r, carry):
                    for jj in range(ROW // 16):
                        sl = pl.ds(jj * 16, 16)
                        ri[r, sl] = ri[r, sl] + rj[r, sl]
                    return carry

                lax.fori_loop(0, 128, row, 0, unroll=4)
                pltpu.make_async_copy(ri, out_hbm.at[pl.ds(e0, 128)],
                                      ws).start()

        start(0, id0, is0, ri0, rj0, gi0, gj0, ws0)
        start(1, id1, is1, ri1, rj1, gi1, gj1, ws1)

        def body(p, carry):
            it = 2 * p
            finish(it, ri0, rj0, gi0, gj0, ws0)
            start(it + 2, id0, is0, ri0, rj0, gi0, gj0, ws0)
            finish(it + 1, ri1, rj1, gi1, gj1, ws1)
            start(it + 3, id1, is1, ri1, rj1, gi1, gj1, ws1)
            return carry

        lax.fori_loop(0, maxp, body, 0)
        pltpu.make_async_copy(ri0, out_hbm.at[pl.ds(0, 128)], ws0).wait()
        pltpu.make_async_copy(ri1, out_hbm.at[pl.ds(0, 128)], ws1).wait()

    return k(ti, tj, src, dst)


def _sc_scatter_add(u, dst, zrows):
    e = u.shape[0]
    n = 10000
    nblk = e // 128
    half = nblk // 2
    maxp = (half // 16 + 2) // 2
    zr = zrows.shape[0]
    mesh = plsc.VectorSubcoreMesh(core_axis_name="c", subcore_axis_name="s")

    @functools.partial(
        pl.kernel,
        out_type=jax.ShapeDtypeStruct((2, n, ROW), F32),
        mesh=mesh,
        scratch_types=[
            pltpu.VMEM((128,), jnp.int32), pltpu.VMEM((128,), jnp.int32),
            pltpu.VMEM((128, ROW), F32), pltpu.VMEM((128, ROW), F32),
            pltpu.VMEM((zr, ROW), F32),
            pltpu.VMEM_SHARED((n, ROW), F32),
            pltpu.SemaphoreType.DMA, pltpu.SemaphoreType.DMA,
            pltpu.SemaphoreType.DMA, pltpu.SemaphoreType.DMA,
            pltpu.SemaphoreType.DMA, pltpu.SemaphoreType.DMA,
        ],
    )
    def k(u_hbm, dst_hbm, z_hbm, out_hbm, ix0, ix1, u0, u1, z_v, acc,
          li0, lu0, li1, lu1, as0, as1):
        c = lax.axis_index("c")
        s = lax.axis_index("s")
        pltpu.sync_copy(z_hbm, z_v)
        nchunk = jnp.where(s == 15, 5, 8)

        def zbody(kk, carry):
            pltpu.sync_copy(z_v, acc.at[pl.ds(s * 640 + kk * zr, zr)])
            return carry

        lax.fori_loop(0, nchunk, zbody, 0)
        plsc.subcore_barrier()

        nb = (half - s + 15) // 16

        def start(it, ix, uv, li, lu, asem):
            @pl.when(jnp.logical_and(it < nb, it >= 2))
            def _():
                pltpu.make_async_copy(uv, acc.at[pl.ds(0, 128)], asem).wait()

            @pl.when(it < nb)
            def _():
                e0 = (c * half + s + it * 16) * 128
                pltpu.make_async_copy(dst_hbm.at[pl.ds(e0, 128)], ix,
                                      li).start()
                pltpu.make_async_copy(u_hbm.at[pl.ds(e0, 128)], uv,
                                      lu).start()

        def process(it, ix, uv, li, lu, asem):
            @pl.when(it < nb)
            def _():
                pltpu.make_async_copy(dst_hbm.at[pl.ds(0, 128)], ix,
                                      li).wait()
                pltpu.make_async_copy(u_hbm.at[pl.ds(0, 128)], uv, lu).wait()
                pltpu.make_async_copy(uv, acc.at[ix], asem).start(add=True)

        start(0, ix0, u0, li0, lu0, as0)
        start(1, ix1, u1, li1, lu1, as1)

        def body(p, carry):
            it = 2 * p
            process(it, ix0, u0, li0, lu0, as0)
            start(it + 2, ix0, u0, li0, lu0, as0)
            process(it + 1, ix1, u1, li1, lu1, as1)
            start(it + 3, ix1, u1, li1, lu1, as1)
            return carry

        lax.fori_loop(0, maxp, body, 0)
        pltpu.make_async_copy(u0, acc.at[pl.ds(0, 128)], as0).wait()
        pltpu.make_async_copy(u1, acc.at[pl.ds(0, 128)], as1).wait()
        plsc.subcore_barrier()

        def wbody(kk, carry):
            r0 = s * 640 + kk * zr
            pltpu.sync_copy(acc.at[pl.ds(r0, zr)], z_v)
            pltpu.sync_copy(z_v, out_hbm.at[c, pl.ds(r0, zr)])
            return carry

        lax.fori_loop(0, nchunk, wbody, 0)

    return k(u, dst, zrows)


def _tc_prep_tables(x, pos, wi, wj, bn):
    n, f = x.shape

    def body(x_ref, pos_ref, wi_ref, wj_ref, ti_ref, tj_ref):
        xb = x_ref[...]
        p = pos_ref[...]
        pad = jnp.zeros((bn, ROW - H - 3), F32)
        ti_ref[...] = jnp.concatenate(
            [jnp.dot(xb, wi_ref[...], preferred_element_type=F32), p, pad], 1)
        tj_ref[...] = jnp.concatenate(
            [jnp.dot(xb, wj_ref[...], preferred_element_type=F32), -p, pad], 1)

    return pl.pallas_call(
        body,
        grid=(n // bn,),
        in_specs=[
            pl.BlockSpec((bn, f), lambda i: (i, 0)),
            pl.BlockSpec((bn, 3), lambda i: (i, 0)),
            pl.BlockSpec((f, H), lambda i: (0, 0)),
            pl.BlockSpec((f, H), lambda i: (0, 0)),
        ],
        out_specs=[
            pl.BlockSpec((bn, ROW), lambda i: (i, 0)),
            pl.BlockSpec((bn, ROW), lambda i: (i, 0)),
        ],
        out_shape=[
            jax.ShapeDtypeStruct((n, ROW), F32),
            jax.ShapeDtypeStruct((n, ROW), F32),
        ],
    )(x, pos, wi, wj)


def _tc_edge_mlp(sarr, ea, we, emw, emb, cmw, cmb, wr, be, with_coord):
    e = sarr.shape[0]
    d = ea.shape[1]

    def body(s_ref, ea_ref, we_ref, emw_ref, emb_ref, cmw_ref, cmb_ref,
             wr_ref, u_ref):
        sb = s_ref[...]
        xpart = sb[:, :H]
        diff = sb[:, H:H + 3]
        radial = jnp.sum(diff * diff, axis=1, keepdims=True)
        pre = (xpart + jnp.dot(ea_ref[...], we_ref[...],
                               preferred_element_type=F32)
               + radial * wr_ref[...])
        er = _silu(pre)
        eij = _silu(jnp.dot(er, emw_ref[...], preferred_element_type=F32)
                    + emb_ref[...])
        if with_coord:
            sc = _silu(jnp.sum(eij * cmw_ref[...], axis=1, keepdims=True)
                       + cmb_ref[...])
            u_ref[...] = jnp.concatenate(
                [eij, diff * sc, jnp.ones((be, 1), F32),
                 jnp.zeros((be, ROW - H - 4), F32)], 1)
        else:
            u_ref[...] = jnp.concatenate(
                [eij, jnp.zeros((be, ROW - H), F32)], 1)

    return pl.pallas_call(
        body,
        grid=(e // be,),
        in_specs=[
            pl.BlockSpec((be, ROW), lambda i: (i, 0)),
            pl.BlockSpec((be, d), lambda i: (i, 0)),
            pl.BlockSpec((d, H), lambda i: (0, 0)),
            pl.BlockSpec((H, H), lambda i: (0, 0)),
            pl.BlockSpec((1, H), lambda i: (0, 0)),
            pl.BlockSpec((1, H), lambda i: (0, 0)),
            pl.BlockSpec((1, 1), lambda i: (0, 0)),
            pl.BlockSpec((1, H), lambda i: (0, 0)),
        ],
        out_specs=pl.BlockSpec((be, ROW), lambda i: (i, 0)),
        out_shape=jax.ShapeDtypeStruct((e, ROW), F32),
    )(sarr, ea, we, emw, emb, cmw, cmb, wr)


def _tc_node0(x, pos, acc0, acc1, w1x, w1e, b1, w2, b2, wi1, wj1, bn):
    n, f = x.shape

    def body(x_ref, pos_ref, a0_ref, a1_ref, w1x_ref, w1e_ref, b1_ref,
             w2_ref, b2_ref, wi_ref, wj_ref, x1_ref, ti_ref, tj_ref):
        a0 = a0_ref[...]
        a1 = a1_ref[...]
        e_agg = a0[:, :H] + a1[:, :H]
        csum = a0[:, H:H + 3] + a1[:, H:H + 3]
        cnt = a0[:, H + 3:H + 4] + a1[:, H + 3:H + 4]
        posn = pos_ref[...] + csum / jnp.maximum(cnt, 1.0)
        h = _silu(jnp.dot(x_ref[...], w1x_ref[...], preferred_element_type=F32)
                  + jnp.dot(e_agg, w1e_ref[...], preferred_element_type=F32)
                  + b1_ref[...])
        x1 = jnp.dot(h, w2_ref[...], preferred_element_type=F32) + b2_ref[...]
        x1_ref[...] = x1
        pad = jnp.zeros((bn, ROW - H - 3), F32)
        ti_ref[...] = jnp.concatenate(
            [jnp.dot(x1, wi_ref[...], preferred_element_type=F32), posn, pad], 1)
        tj_ref[...] = jnp.concatenate(
            [jnp.dot(x1, wj_ref[...], preferred_element_type=F32), -posn, pad], 1)

    c0 = pl.BlockSpec((H, H), lambda i: (0, 0))
    cb = pl.BlockSpec((1, H), lambda i: (0, 0))
    return pl.pallas_call(
        body,
        grid=(n // bn,),
        in_specs=[
            pl.BlockSpec((bn, f), lambda i: (i, 0)),
            pl.BlockSpec((bn, 3), lambda i: (i, 0)),
            pl.BlockSpec((bn, ROW), lambda i: (i, 0)),
            pl.BlockSpec((bn, ROW), lambda i: (i, 0)),
            pl.BlockSpec((f, H), lambda i: (0, 0)),
            c0, cb, c0, cb, c0, c0,
        ],
        out_specs=[
            pl.BlockSpec((bn, H), lambda i: (i, 0)),
            pl.BlockSpec((bn, ROW), lambda i: (i, 0)),
            pl.BlockSpec((bn, ROW), lambda i: (i, 0)),
        ],
        out_shape=[
            jax.ShapeDtypeStruct((n, H), F32),
            jax.ShapeDtypeStruct((n, ROW), F32),
            jax.ShapeDtypeStruct((n, ROW), F32),
        ],
    )(x, pos, acc0, acc1, w1x, w1e, b1, w2, b2, wi1, wj1)


def _tc_node1_pool(x1, acc0, acc1, batch_col, w1x, w1e, b1, w2, b2, bn):
    n = x1.shape[0]
    g = 64

    def body(x_ref, a0_ref, a1_ref, b_ref, w1x_ref, w1e_ref, b1_ref,
             w2_ref, b2_ref, out_ref):
        e_agg = a0_ref[:, :H] + a1_ref[:, :H]
        h = _silu(jnp.dot(x_ref[...], w1x_ref[...], preferred_element_type=F32)
                  + jnp.dot(e_agg, w1e_ref[...], preferred_element_type=F32)
                  + b1_ref[...])
        x2 = jnp.dot(h, w2_ref[...], preferred_element_type=F32) + b2_ref[...]
        gid = lax.broadcasted_iota(jnp.int32, (1, g), 1)
        oh = (b_ref[...] == gid).astype(F32)
        feat = jnp.concatenate(
            [x2, jnp.ones((bn, 1), F32), jnp.zeros((bn, 7), F32)], 1)
        contrib = lax.dot_general(oh, feat, (((0,), (0,)), ((), ())),
                                  preferred_element_type=F32)

        @pl.when(pl.program_id(0) == 0)
        def _():
            out_ref[...] = contrib

        @pl.when(pl.program_id(0) != 0)
        def _():
            out_ref[...] = out_ref[...] + contrib

    c0 = pl.BlockSpec((H, H), lambda i: (0, 0))
    cb = pl.BlockSpec((1, H), lambda i: (0, 0))
    return pl.pallas_call(
        body,
        grid=(n // bn,),
        in_specs=[
            pl.BlockSpec((bn, H), lambda i: (i, 0)),
            pl.BlockSpec((bn, ROW), lambda i: (i, 0)),
            pl.BlockSpec((bn, ROW), lambda i: (i, 0)),
            pl.BlockSpec((bn, 1), lambda i: (i, 0)),
            c0, c0, cb, c0, cb,
        ],
        out_specs=pl.BlockSpec((g, H + 8), lambda i: (0, 0)),
        out_shape=jax.ShapeDtypeStruct((g, H + 8), F32),
    )(x1, acc0, acc1, batch_col, w1x, w1e, b1, w2, b2)


def _tc_final(pooled, w1, b1, w2, b2):
    g = pooled.shape[0]
    out_c = w2.shape[1]

    def body(p_ref, w1_ref, b1_ref, w2_ref, b2_ref, o_ref):
        p = p_ref[...]
        xg = p[:, :H] / jnp.maximum(p[:, H:H + 1], 1.0)
        m = jnp.max(xg, axis=1, keepdims=True)
        z = xg - m
        xl = z - jnp.log(jnp.sum(jnp.exp(z), axis=1, keepdims=True))
        hh = jnp.maximum(
            jnp.dot(xl, w1_ref[...], preferred_element_type=F32) + b1_ref[...],
            0.0)
        o_ref[...] = (jnp.dot(hh, w2_ref[...], preferred_element_type=F32)
                      + b2_ref[...])

    return pl.pallas_call(
        body,
        out_shape=jax.ShapeDtypeStruct((g, out_c), F32),
    )(pooled, w1, b1, w2, b2)


def kernel(x, pos, edge_attr, edge_index, batch, params):
    n, f = x.shape
    d = edge_attr.shape[1]
    src = edge_index[0]
    dst = edge_index[1]

    p0, p1, po = params['l0'], params['l1'], params['out']
    w0 = p0['mlp_W']
    wi0, wj0 = w0[:f], w0[f:2 * f]
    we0, wr0 = w0[2 * f:2 * f + d], w0[2 * f + d].reshape(1, H)
    w1 = p1['mlp_W']
    wi1, wj1 = w1[:H], w1[H:2 * H]
    we1, wr1 = w1[2 * H:2 * H + d], w1[2 * H + d].reshape(1, H)

    zrows = jnp.zeros((80, ROW), F32)
    batch_col = batch.reshape(n, 1)

    ti0, tj0 = _tc_prep_tables(x, pos, wi0, wj0, bn=2000)

    s0 = _sc_gather_sum(ti0, tj0, src, dst)
    u0 = _tc_edge_mlp(s0, edge_attr, we0, p0['em_W'], p0['em_b'].reshape(1, H),
                      p0['cm_W'].reshape(1, H), p0['cm_b'].reshape(1, 1),
                      wr0, be=2000, with_coord=True)
    acc = _sc_scatter_add(u0, dst, zrows)
    x1, ti1, tj1 = _tc_node0(
        x, pos, acc[0], acc[1],
        p0['nm_W1'][:f], p0['nm_W1'][f:], p0['nm_b1'].reshape(1, H),
        p0['nm_W2'], p0['nm_b2'].reshape(1, H), wi1, wj1, bn=2000)

    s1 = _sc_gather_sum(ti1, tj1, src, dst)
    u1 = _tc_edge_mlp(s1, edge_attr, we1, p1['em_W'], p1['em_b'].reshape(1, H),
                      p1['cm_W'].reshape(1, H), p1['cm_b'].reshape(1, 1),
                      wr1, be=2000, with_coord=False)
    acc1 = _sc_scatter_add(u1, dst, zrows)
    pooled = _tc_node1_pool(
        x1, acc1[0], acc1[1], batch_col,
        p1['nm_W1'][:H], p1['nm_W1'][H:], p1['nm_b1'].reshape(1, H),
        p1['nm_W2'], p1['nm_b2'].reshape(1, H), bn=2000)

    return _tc_final(pooled, po['W1'], po['b1'].reshape(1, H),
                     po['W2'], po['b2'].reshape(1, po['W2'].shape[1]))


# speedup vs baseline: 5.4037x; 1.3206x over previous
import functools

import jax
import jax.numpy as jnp
from jax import lax
from jax.experimental import pallas as pl
from jax.experimental.pallas import tpu as pltpu
from jax.experimental.pallas import tpu_sc as plsc

F32 = jnp.float32
ROW = 128
AW = 72
H = 64


def _silu(v):
    return v * (1.0 / (1.0 + jnp.exp(-v)))


def _sc_gather_sum(ti, tj, src, dst):
    e = src.shape[0]
    BLK = 64
    nblk = e // BLK
    maxp = (nblk // 32 + 3) // 3
    mesh = plsc.VectorSubcoreMesh(core_axis_name="c", subcore_axis_name="s")

    @functools.partial(
        pl.kernel,
        out_type=jax.ShapeDtypeStruct((e, ROW), F32),
        mesh=mesh,
        scratch_types=[
            [pltpu.VMEM((BLK,), jnp.int32)] * 3,
            [pltpu.VMEM((BLK,), jnp.int32)] * 3,
            [pltpu.VMEM((BLK, ROW), F32)] * 3,
            [pltpu.VMEM((BLK, ROW), F32)] * 3,
            [pltpu.VMEM((BLK, ROW), F32)] * 3,
            [pltpu.SemaphoreType.DMA] * 3,
            [pltpu.SemaphoreType.DMA] * 3,
            [pltpu.SemaphoreType.DMA] * 3,
            [pltpu.SemaphoreType.DMA] * 3,
            [pltpu.SemaphoreType.DMA] * 3,
        ],
        compiler_params=pltpu.CompilerParams(use_tc_tiling_on_sc=False),
    )
    def k(ti_hbm, tj_hbm, src_hbm, dst_hbm, out_hbm,
          idd, ids, ri, rj, ro, sid, sis, sgi, sgj, sw):
        c = lax.axis_index("c")
        s = lax.axis_index("s")
        wid = s * 2 + c
        nb = (nblk - wid + 31) // 32

        def start_idx(q, it):
            @pl.when(it < nb)
            def _():
                e0 = (wid + it * 32) * BLK
                pltpu.make_async_copy(dst_hbm.at[pl.ds(e0, BLK)], idd[q],
                                      sid[q]).start()
                pltpu.make_async_copy(src_hbm.at[pl.ds(e0, BLK)], ids[q],
                                      sis[q]).start()

        def start_gather(q, it):
            @pl.when(it < nb)
            def _():
                pltpu.make_async_copy(dst_hbm.at[pl.ds(0, BLK)], idd[q],
                                      sid[q]).wait()
                pltpu.make_async_copy(src_hbm.at[pl.ds(0, BLK)], ids[q],
                                      sis[q]).wait()
                pltpu.make_async_copy(ti_hbm.at[idd[q]], ri[q], sgi[q]).start()
                pltpu.make_async_copy(tj_hbm.at[ids[q]], rj[q], sgj[q]).start()

        def finish(q, it):
            @pl.when(it < nb)
            def _():
                e0 = (wid + it * 32) * BLK
                pltpu.make_async_copy(ti_hbm.at[pl.ds(0, BLK)], ri[q],
                                      sgi[q]).wait()
                pltpu.make_async_copy(tj_hbm.at[pl.ds(0, BLK)], rj[q],
                                      sgj[q]).wait()

                @pl.when(it >= 3)
                def _():
                    pltpu.make_async_copy(ro[q], out_hbm.at[pl.ds(0, BLK)],
                                          sw[q]).wait()

                def row(r, carry):
                    for jj in range(5):
                        sl = pl.ds(jj * 16, 16)
                        ro[q][r, sl] = ri[q][r, sl] + rj[q][r, sl]
                    return carry

                lax.fori_loop(0, BLK, row, 0, unroll=4)
                pltpu.make_async_copy(ro[q], out_hbm.at[pl.ds(e0, BLK)],
                                      sw[q]).start()

        start_idx(0, 0)
        start_idx(1, 1)
        start_idx(2, 2)
        start_gather(0, 0)
        start_gather(1, 1)

        def body(p, carry):
            for q in range(3):
                it = 3 * p + q
                finish(q, it)
                start_idx(q, it + 3)
                start_gather((q + 2) % 3, it + 2)
            return carry

        lax.fori_loop(0, maxp, body, 0)
        for q in range(3):
            pltpu.make_async_copy(ro[q], out_hbm.at[pl.ds(0, BLK)],
                                  sw[q]).wait()

    return k(ti, tj, src, dst)


def _sc_scatter_add(u, dst, zrows):
    e = u.shape[0]
    n = 10000
    nblk = e // 128
    half = nblk // 2
    maxp = (half // 16 + 2) // 2
    zr = zrows.shape[0]
    mesh = plsc.VectorSubcoreMesh(core_axis_name="c", subcore_axis_name="s")

    @functools.partial(
        pl.kernel,
        out_type=jax.ShapeDtypeStruct((2, n, AW), F32),
        mesh=mesh,
        scratch_types=[
            pltpu.VMEM((128,), jnp.int32), pltpu.VMEM((128,), jnp.int32),
            pltpu.VMEM((128, AW), F32), pltpu.VMEM((128, AW), F32),
            pltpu.VMEM((zr, AW), F32),
            pltpu.VMEM_SHARED((n, AW), F32),
            pltpu.SemaphoreType.DMA, pltpu.SemaphoreType.DMA,
            pltpu.SemaphoreType.DMA, pltpu.SemaphoreType.DMA,
            pltpu.SemaphoreType.DMA, pltpu.SemaphoreType.DMA,
        ],
        compiler_params=pltpu.CompilerParams(use_tc_tiling_on_sc=False),
    )
    def k(u_hbm, dst_hbm, z_hbm, out_hbm, ix0, ix1, u0, u1, z_v, acc,
          li0, lu0, li1, lu1, as0, as1):
        c = lax.axis_index("c")
        s = lax.axis_index("s")
        pltpu.sync_copy(z_hbm, z_v)
        nchunk = jnp.where(s == 15, 5, 8)

        def zbody(kk, carry):
            pltpu.sync_copy(z_v, acc.at[pl.ds(s * 640 + kk * zr, zr)])
            return carry

        lax.fori_loop(0, nchunk, zbody, 0)
        plsc.subcore_barrier()

        nb = (half - s + 15) // 16

        def start(it, ix, uv, li, lu, asem):
            @pl.when(jnp.logical_and(it < nb, it >= 2))
            def _():
                pltpu.make_async_copy(uv, acc.at[pl.ds(0, 128)], asem).wait()

            @pl.when(it < nb)
            def _():
                e0 = (c * half + s + it * 16) * 128
                pltpu.make_async_copy(dst_hbm.at[pl.ds(e0, 128)], ix,
                                      li).start()
                pltpu.make_async_copy(
                    u_hbm.at[pl.ds(e0, 128), pl.ds(0, AW)], uv, lu).start()

        def process(it, ix, uv, li, lu, asem):
            @pl.when(it < nb)
            def _():
                pltpu.make_async_copy(dst_hbm.at[pl.ds(0, 128)], ix,
                                      li).wait()
                pltpu.make_async_copy(u_hbm.at[pl.ds(0, 128), pl.ds(0, AW)],
                                      uv, lu).wait()
                pltpu.make_async_copy(uv, acc.at[ix], asem).start(add=True)

        start(0, ix0, u0, li0, lu0, as0)
        start(1, ix1, u1, li1, lu1, as1)

        def body(p, carry):
            it = 2 * p
            process(it, ix0, u0, li0, lu0, as0)
            start(it + 2, ix0, u0, li0, lu0, as0)
            process(it + 1, ix1, u1, li1, lu1, as1)
            start(it + 3, ix1, u1, li1, lu1, as1)
            return carry

        lax.fori_loop(0, maxp, body, 0)
        pltpu.make_async_copy(u0, acc.at[pl.ds(0, 128)], as0).wait()
        pltpu.make_async_copy(u1, acc.at[pl.ds(0, 128)], as1).wait()
        plsc.subcore_barrier()

        def wbody(kk, carry):
            r0 = s * 640 + kk * zr
            pltpu.sync_copy(acc.at[pl.ds(r0, zr)], z_v)
            pltpu.sync_copy(z_v, out_hbm.at[c, pl.ds(r0, zr)])
            return carry

        lax.fori_loop(0, nchunk, wbody, 0)

    return k(u, dst, zrows)


def _tc_prep_tables(x, pos, wi, wj, bn):
    n, f = x.shape

    def body(x_ref, pos_ref, wi_ref, wj_ref, ti_ref, tj_ref):
        xb = x_ref[...]
        p = pos_ref[...]
        pad = jnp.zeros((bn, ROW - H - 3), F32)
        ti_ref[...] = jnp.concatenate(
            [jnp.dot(xb, wi_ref[...], preferred_element_type=F32), p, pad], 1)
        tj_ref[...] = jnp.concatenate(
            [jnp.dot(xb, wj_ref[...], preferred_element_type=F32), -p, pad], 1)

    return pl.pallas_call(
        body,
        grid=(n // bn,),
        in_specs=[
            pl.BlockSpec((bn, f), lambda i: (i, 0)),
            pl.BlockSpec((bn, 3), lambda i: (i, 0)),
            pl.BlockSpec((f, H), lambda i: (0, 0)),
            pl.BlockSpec((f, H), lambda i: (0, 0)),
        ],
        out_specs=[
            pl.BlockSpec((bn, ROW), lambda i: (i, 0)),
            pl.BlockSpec((bn, ROW), lambda i: (i, 0)),
        ],
        out_shape=[
            jax.ShapeDtypeStruct((n, ROW), F32),
            jax.ShapeDtypeStruct((n, ROW), F32),
        ],
    )(x, pos, wi, wj)


def _tc_edge_mlp(sarr, ea, we, emw, emb, cmw, cmb, wr, be, with_coord):
    e = sarr.shape[0]
    d = ea.shape[1]

    def body(s_ref, ea_ref, we_ref, emw_ref, emb_ref, cmw_ref, cmb_ref,
             wr_ref, u_ref):
        sb = s_ref[...]
        xpart = sb[:, :H]
        diff = sb[:, H:H + 3]
        radial = jnp.sum(diff * diff, axis=1, keepdims=True)
        pre = (xpart + jnp.dot(ea_ref[...], we_ref[...],
                               preferred_element_type=F32)
               + radial * wr_ref[...])
        er = _silu(pre)
        eij = _silu(jnp.dot(er, emw_ref[...], preferred_element_type=F32)
                    + emb_ref[...])
        if with_coord:
            sc = _silu(jnp.sum(eij * cmw_ref[...], axis=1, keepdims=True)
                       + cmb_ref[...])
            u_ref[...] = jnp.concatenate(
                [eij, diff * sc, jnp.ones((be, 1), F32),
                 jnp.zeros((be, ROW - H - 4), F32)], 1)
        else:
            u_ref[...] = jnp.concatenate(
                [eij, jnp.zeros((be, ROW - H), F32)], 1)

    return pl.pallas_call(
        body,
        grid=(e // be,),
        in_specs=[
            pl.BlockSpec((be, ROW), lambda i: (i, 0)),
            pl.BlockSpec((be, d), lambda i: (i, 0)),
            pl.BlockSpec((d, H), lambda i: (0, 0)),
            pl.BlockSpec((H, H), lambda i: (0, 0)),
            pl.BlockSpec((1, H), lambda i: (0, 0)),
            pl.BlockSpec((1, H), lambda i: (0, 0)),
            pl.BlockSpec((1, 1), lambda i: (0, 0)),
            pl.BlockSpec((1, H), lambda i: (0, 0)),
        ],
        out_specs=pl.BlockSpec((be, ROW), lambda i: (i, 0)),
        out_shape=jax.ShapeDtypeStruct((e, ROW), F32),
    )(sarr, ea, we, emw, emb, cmw, cmb, wr)


def _tc_node0(x, pos, acc0, acc1, w1x, w1e, b1, w2, b2, wi1, wj1, bn):
    n, f = x.shape

    def body(x_ref, pos_ref, a0_ref, a1_ref, w1x_ref, w1e_ref, b1_ref,
             w2_ref, b2_ref, wi_ref, wj_ref, x1_ref, ti_ref, tj_ref):
        a0 = a0_ref[...]
        a1 = a1_ref[...]
        e_agg = a0[:, :H] + a1[:, :H]
        csum = a0[:, H:H + 3] + a1[:, H:H + 3]
        cnt = a0[:, H + 3:H + 4] + a1[:, H + 3:H + 4]
        posn = pos_ref[...] + csum / jnp.maximum(cnt, 1.0)
        h = _silu(jnp.dot(x_ref[...], w1x_ref[...], preferred_element_type=F32)
                  + jnp.dot(e_agg, w1e_ref[...], preferred_element_type=F32)
                  + b1_ref[...])
        x1 = jnp.dot(h, w2_ref[...], preferred_element_type=F32) + b2_ref[...]
        x1_ref[...] = x1
        pad = jnp.zeros((bn, ROW - H - 3), F32)
        ti_ref[...] = jnp.concatenate(
            [jnp.dot(x1, wi_ref[...], preferred_element_type=F32), posn, pad], 1)
        tj_ref[...] = jnp.concatenate(
            [jnp.dot(x1, wj_ref[...], preferred_element_type=F32), -posn, pad], 1)

    c0 = pl.BlockSpec((H, H), lambda i: (0, 0))
    cb = pl.BlockSpec((1, H), lambda i: (0, 0))
    return pl.pallas_call(
        body,
        grid=(n // bn,),
        in_specs=[
            pl.BlockSpec((bn, f), lambda i: (i, 0)),
            pl.BlockSpec((bn, 3), lambda i: (i, 0)),
            pl.BlockSpec((bn, AW), lambda i: (i, 0)),
            pl.BlockSpec((bn, AW), lambda i: (i, 0)),
            pl.BlockSpec((f, H), lambda i: (0, 0)),
            c0, cb, c0, cb, c0, c0,
        ],
        out_specs=[
            pl.BlockSpec((bn, H), lambda i: (i, 0)),
            pl.BlockSpec((bn, ROW), lambda i: (i, 0)),
            pl.BlockSpec((bn, ROW), lambda i: (i, 0)),
        ],
        out_shape=[
            jax.ShapeDtypeStruct((n, H), F32),
            jax.ShapeDtypeStruct((n, ROW), F32),
            jax.ShapeDtypeStruct((n, ROW), F32),
        ],
    )(x, pos, acc0, acc1, w1x, w1e, b1, w2, b2, wi1, wj1)


def _tc_node1_pool(x1, acc0, acc1, batch_col, w1x, w1e, b1, w2, b2, bn):
    n = x1.shape[0]
    g = 64

    def body(x_ref, a0_ref, a1_ref, b_ref, w1x_ref, w1e_ref, b1_ref,
             w2_ref, b2_ref, out_ref):
        e_agg = a0_ref[:, :H] + a1_ref[:, :H]
        h = _silu(jnp.dot(x_ref[...], w1x_ref[...], preferred_element_type=F32)
                  + jnp.dot(e_agg, w1e_ref[...], preferred_element_type=F32)
                  + b1_ref[...])
        x2 = jnp.dot(h, w2_ref[...], preferred_element_type=F32) + b2_ref[...]
        gid = lax.broadcasted_iota(jnp.int32, (1, g), 1)
        oh = (b_ref[...] == gid).astype(F32)
        feat = jnp.concatenate(
            [x2, jnp.ones((bn, 1), F32), jnp.zeros((bn, 7), F32)], 1)
        contrib = lax.dot_general(oh, feat, (((0,), (0,)), ((), ())),
                                  preferred_element_type=F32)

        @pl.when(pl.program_id(0) == 0)
        def _():
            out_ref[...] = contrib

        @pl.when(pl.program_id(0) != 0)
        def _():
            out_ref[...] = out_ref[...] + contrib

    c0 = pl.BlockSpec((H, H), lambda i: (0, 0))
    cb = pl.BlockSpec((1, H), lambda i: (0, 0))
    return pl.pallas_call(
        body,
        grid=(n // bn,),
        in_specs=[
            pl.BlockSpec((bn, H), lambda i: (i, 0)),
            pl.BlockSpec((bn, AW), lambda i: (i, 0)),
            pl.BlockSpec((bn, AW), lambda i: (i, 0)),
            pl.BlockSpec((bn, 1), lambda i: (i, 0)),
            c0, c0, cb, c0, cb,
        ],
        out_specs=pl.BlockSpec((g, H + 8), lambda i: (0, 0)),
        out_shape=jax.ShapeDtypeStruct((g, H + 8), F32),
    )(x1, acc0, acc1, batch_col, w1x, w1e, b1, w2, b2)


def _tc_final(pooled, w1, b1, w2, b2):
    g = pooled.shape[0]
    out_c = w2.shape[1]

    def body(p_ref, w1_ref, b1_ref, w2_ref, b2_ref, o_ref):
        p = p_ref[...]
        xg = p[:, :H] / jnp.maximum(p[:, H:H + 1], 1.0)
        m = jnp.max(xg, axis=1, keepdims=True)
        z = xg - m
        xl = z - jnp.log(jnp.sum(jnp.exp(z), axis=1, keepdims=True))
        hh = jnp.maximum(
            jnp.dot(xl, w1_ref[...], preferred_element_type=F32) + b1_ref[...],
            0.0)
        o_ref[...] = (jnp.dot(hh, w2_ref[...], preferred_element_type=F32)
                      + b2_ref[...])

    return pl.pallas_call(
        body,
        out_shape=jax.ShapeDtypeStruct((g, out_c), F32),
    )(pooled, w1, b1, w2, b2)


def kernel(x, pos, edge_attr, edge_index, batch, params):
    n, f = x.shape
    d = edge_attr.shape[1]
    src = edge_index[0]
    dst = edge_index[1]

    p0, p1, po = params['l0'], params['l1'], params['out']
    w0 = p0['mlp_W']
    wi0, wj0 = w0[:f], w0[f:2 * f]
    we0, wr0 = w0[2 * f:2 * f + d], w0[2 * f + d].reshape(1, H)
    w1 = p1['mlp_W']
    wi1, wj1 = w1[:H], w1[H:2 * H]
    we1, wr1 = w1[2 * H:2 * H + d], w1[2 * H + d].reshape(1, H)

    zrows = jnp.zeros((80, AW), F32)
    batch_col = batch.reshape(n, 1)

    ti0, tj0 = _tc_prep_tables(x, pos, wi0, wj0, bn=2000)

    s0 = _sc_gather_sum(ti0, tj0, src, dst)
    u0 = _tc_edge_mlp(s0, edge_attr, we0, p0['em_W'], p0['em_b'].reshape(1, H),
                      p0['cm_W'].reshape(1, H), p0['cm_b'].reshape(1, 1),
                      wr0, be=2000, with_coord=True)
    acc = _sc_scatter_add(u0, dst, zrows)
    x1, ti1, tj1 = _tc_node0(
        x, pos, acc[0], acc[1],
        p0['nm_W1'][:f], p0['nm_W1'][f:], p0['nm_b1'].reshape(1, H),
        p0['nm_W2'], p0['nm_b2'].reshape(1, H), wi1, wj1, bn=2000)

    s1 = _sc_gather_sum(ti1, tj1, src, dst)
    u1 = _tc_edge_mlp(s1, edge_attr, we1, p1['em_W'], p1['em_b'].reshape(1, H),
                      p1['cm_W'].reshape(1, H), p1['cm_b'].reshape(1, 1),
                      wr1, be=2000, with_coord=False)
    acc1 = _sc_scatter_add(u1, dst, zrows)
    pooled = _tc_node1_pool(
        x1, acc1[0], acc1[1], batch_col,
        p1['nm_W1'][:H], p1['nm_W1'][H:], p1['nm_b1'].reshape(1, H),
        p1['nm_W2'], p1['nm_b2'].reshape(1, H), bn=2000)

    return _tc_final(pooled, po['W1'], po['b1'].reshape(1, H),
                     po['W2'], po['b2'].reshape(1, po['W2'].shape[1]))


# speedup vs baseline: 6.4094x; 1.1861x over previous
import functools

import jax
import jax.numpy as jnp
from jax import lax
from jax.experimental import pallas as pl
from jax.experimental.pallas import tpu as pltpu
from jax.experimental.pallas import tpu_sc as plsc

F32 = jnp.float32
ROW = 128
AW = 72
H = 64


def _silu(v):
    return v * (1.0 / (1.0 + jnp.exp(-v)))


def _sc_gather_sum(ti, tj, src, dst):
    e = src.shape[0]
    BLK = 64
    nblk = e // BLK
    maxp = (nblk // 32 + 3) // 3
    mesh = plsc.VectorSubcoreMesh(core_axis_name="c", subcore_axis_name="s")

    @functools.partial(
        pl.kernel,
        out_type=jax.ShapeDtypeStruct((e, ROW), F32),
        mesh=mesh,
        scratch_types=[
            [pltpu.VMEM((BLK,), jnp.int32)] * 3,
            [pltpu.VMEM((BLK,), jnp.int32)] * 3,
            [pltpu.VMEM((BLK, ROW), F32)] * 3,
            [pltpu.VMEM((BLK, ROW), F32)] * 3,
            [pltpu.VMEM((BLK, ROW), F32)] * 3,
            [pltpu.SemaphoreType.DMA] * 3,
            [pltpu.SemaphoreType.DMA] * 3,
            [pltpu.SemaphoreType.DMA] * 3,
            [pltpu.SemaphoreType.DMA] * 3,
            [pltpu.SemaphoreType.DMA] * 3,
        ],
        compiler_params=pltpu.CompilerParams(use_tc_tiling_on_sc=False),
    )
    def k(ti_hbm, tj_hbm, src_hbm, dst_hbm, out_hbm,
          idd, ids, ri, rj, ro, sid, sis, sgi, sgj, sw):
        c = lax.axis_index("c")
        s = lax.axis_index("s")
        wid = s * 2 + c
        nb = (nblk - wid + 31) // 32

        def start_idx(q, it):
            @pl.when(it < nb)
            def _():
                e0 = (wid + it * 32) * BLK
                pltpu.make_async_copy(dst_hbm.at[pl.ds(e0, BLK)], idd[q],
                                      sid[q]).start()
                pltpu.make_async_copy(src_hbm.at[pl.ds(e0, BLK)], ids[q],
                                      sis[q]).start()

        def start_gather(q, it):
            @pl.when(it < nb)
            def _():
                pltpu.make_async_copy(dst_hbm.at[pl.ds(0, BLK)], idd[q],
                                      sid[q]).wait()
                pltpu.make_async_copy(src_hbm.at[pl.ds(0, BLK)], ids[q],
                                      sis[q]).wait()
                pltpu.make_async_copy(ti_hbm.at[idd[q]], ri[q], sgi[q]).start()
                pltpu.make_async_copy(tj_hbm.at[ids[q]], rj[q], sgj[q]).start()

        def finish(q, it):
            @pl.when(it < nb)
            def _():
                e0 = (wid + it * 32) * BLK
                pltpu.make_async_copy(ti_hbm.at[pl.ds(0, BLK)], ri[q],
                                      sgi[q]).wait()
                pltpu.make_async_copy(tj_hbm.at[pl.ds(0, BLK)], rj[q],
                                      sgj[q]).wait()

                @pl.when(it >= 3)
                def _():
                    pltpu.make_async_copy(ro[q], out_hbm.at[pl.ds(0, BLK)],
                                          sw[q]).wait()

                def row(r, carry):
                    for jj in range(5):
                        sl = pl.ds(jj * 16, 16)
                        ro[q][r, sl] = ri[q][r, sl] + rj[q][r, sl]
                    return carry

                lax.fori_loop(0, BLK, row, 0, unroll=4)
                pltpu.make_async_copy(ro[q], out_hbm.at[pl.ds(e0, BLK)],
                                      sw[q]).start()

        start_idx(0, 0)
        start_idx(1, 1)
        start_idx(2, 2)
        start_gather(0, 0)
        start_gather(1, 1)

        def body(p, carry):
            for q in range(3):
                it = 3 * p + q
                finish(q, it)
                start_idx(q, it + 3)
                start_gather((q + 2) % 3, it + 2)
            return carry

        lax.fori_loop(0, maxp, body, 0)
        for q in range(3):
            pltpu.make_async_copy(ro[q], out_hbm.at[pl.ds(0, BLK)],
                                  sw[q]).wait()

    return k(ti, tj, src, dst)


def _sc_scatter_add(u, dst, zrows):
    e = u.shape[0]
    n = 10000
    nblk = e // 128
    half = nblk // 2
    maxp = (half // 16 + 2) // 2
    zr = zrows.shape[0]
    mesh = plsc.VectorSubcoreMesh(core_axis_name="c", subcore_axis_name="s")

    @functools.partial(
        pl.kernel,
        out_type=jax.ShapeDtypeStruct((2, n, AW), F32),
        mesh=mesh,
        scratch_types=[
            pltpu.VMEM((128,), jnp.int32), pltpu.VMEM((128,), jnp.int32),
            pltpu.VMEM((128, AW), F32), pltpu.VMEM((128, AW), F32),
            pltpu.VMEM((zr, AW), F32),
            pltpu.VMEM_SHARED((n, AW), F32),
            pltpu.SemaphoreType.DMA, pltpu.SemaphoreType.DMA,
            pltpu.SemaphoreType.DMA, pltpu.SemaphoreType.DMA,
            pltpu.SemaphoreType.DMA, pltpu.SemaphoreType.DMA,
        ],
        compiler_params=pltpu.CompilerParams(use_tc_tiling_on_sc=False),
    )
    def k(u_hbm, dst_hbm, z_hbm, out_hbm, ix0, ix1, u0, u1, z_v, acc,
          li0, lu0, li1, lu1, as0, as1):
        c = lax.axis_index("c")
        s = lax.axis_index("s")
        pltpu.sync_copy(z_hbm, z_v)
        nchunk = jnp.where(s == 15, 5, 8)

        def zbody(kk, carry):
            pltpu.sync_copy(z_v, acc.at[pl.ds(s * 640 + kk * zr, zr)])
            return carry

        lax.fori_loop(0, nchunk, zbody, 0)
        plsc.subcore_barrier()

        nb = (half - s + 15) // 16

        def start(it, ix, uv, li, lu, asem):
            @pl.when(jnp.logical_and(it < nb, it >= 2))
            def _():
                pltpu.make_async_copy(uv, acc.at[pl.ds(0, 128)], asem).wait()

            @pl.when(it < nb)
            def _():
                e0 = (c * half + s + it * 16) * 128
                pltpu.make_async_copy(dst_hbm.at[pl.ds(e0, 128)], ix,
                                      li).start()
                pltpu.make_async_copy(
                    u_hbm.at[pl.ds(e0, 128), pl.ds(0, AW)], uv, lu).start()

        def process(it, ix, uv, li, lu, asem):
            @pl.when(it < nb)
            def _():
                pltpu.make_async_copy(dst_hbm.at[pl.ds(0, 128)], ix,
                                      li).wait()
                pltpu.make_async_copy(u_hbm.at[pl.ds(0, 128), pl.ds(0, AW)],
                                      uv, lu).wait()
                pltpu.make_async_copy(uv, acc.at[ix], asem).start(add=True)

        start(0, ix0, u0, li0, lu0, as0)
        start(1, ix1, u1, li1, lu1, as1)

        def body(p, carry):
            it = 2 * p
            process(it, ix0, u0, li0, lu0, as0)
            start(it + 2, ix0, u0, li0, lu0, as0)
            process(it + 1, ix1, u1, li1, lu1, as1)
            start(it + 3, ix1, u1, li1, lu1, as1)
            return carry

        lax.fori_loop(0, maxp, body, 0)
        pltpu.make_async_copy(u0, acc.at[pl.ds(0, 128)], as0).wait()
        pltpu.make_async_copy(u1, acc.at[pl.ds(0, 128)], as1).wait()
        plsc.subcore_barrier()

        def wbody(kk, carry):
            r0 = s * 640 + kk * zr
            pltpu.sync_copy(acc.at[pl.ds(r0, zr)], z_v)
            pltpu.sync_copy(z_v, out_hbm.at[c, pl.ds(r0, zr)])
            return carry

        lax.fori_loop(0, nchunk, wbody, 0)

    return k(u, dst, zrows)


def _tc_prep_tables(x, pos, wi, wj, bn):
    n, f = x.shape

    def body(x_ref, pos_ref, wi_ref, wj_ref, ti_ref, tj_ref):
        xb = x_ref[...]
        p = pos_ref[...]
        pad = jnp.zeros((bn, ROW - H - 3), F32)
        ti_ref[...] = jnp.concatenate(
            [jnp.dot(xb, wi_ref[...], preferred_element_type=F32), p, pad], 1)
        tj_ref[...] = jnp.concatenate(
            [jnp.dot(xb, wj_ref[...], preferred_element_type=F32), -p, pad], 1)

    return pl.pallas_call(
        body,
        grid=(n // bn,),
        in_specs=[
            pl.BlockSpec((bn, f), lambda i: (i, 0)),
            pl.BlockSpec((bn, 3), lambda i: (i, 0)),
            pl.BlockSpec((f, H), lambda i: (0, 0)),
            pl.BlockSpec((f, H), lambda i: (0, 0)),
        ],
        out_specs=[
            pl.BlockSpec((bn, ROW), lambda i: (i, 0)),
            pl.BlockSpec((bn, ROW), lambda i: (i, 0)),
        ],
        out_shape=[
            jax.ShapeDtypeStruct((n, ROW), F32),
            jax.ShapeDtypeStruct((n, ROW), F32),
        ],
    )(x, pos, wi, wj)


def _tc_edge_mlp(sarr, ea_t, we, emw, emb, cmw, cmb, wr, be, with_coord):
    e = sarr.shape[0]
    d = ea_t.shape[0]

    def body(s_ref, ea_ref, we_ref, emw_ref, emb_ref, cmw_ref, cmb_ref,
             wr_ref, u_ref):
        sb = s_ref[...]
        xpart = sb[:, :H]
        diff = sb[:, H:H + 3]
        radial = jnp.sum(diff * diff, axis=1, keepdims=True)
        pre = (xpart
               + lax.dot_general(ea_ref[...], we_ref[...],
                                 (((0,), (0,)), ((), ())),
                                 preferred_element_type=F32)
               + radial * wr_ref[...])
        er = _silu(pre)
        eij = _silu(jnp.dot(er, emw_ref[...], preferred_element_type=F32)
                    + emb_ref[...])
        if with_coord:
            sc = _silu(jnp.sum(eij * cmw_ref[...], axis=1, keepdims=True)
                       + cmb_ref[...])
            u_ref[...] = jnp.concatenate(
                [eij, diff * sc, jnp.ones((be, 1), F32),
                 jnp.zeros((be, ROW - H - 4), F32)], 1)
        else:
            u_ref[...] = jnp.concatenate([eij, eij], 1)

    return pl.pallas_call(
        body,
        grid=(e // be,),
        in_specs=[
            pl.BlockSpec((be, ROW), lambda i: (i, 0)),
            pl.BlockSpec((d, be), lambda i: (0, i)),
            pl.BlockSpec((d, H), lambda i: (0, 0)),
            pl.BlockSpec((H, H), lambda i: (0, 0)),
            pl.BlockSpec((1, H), lambda i: (0, 0)),
            pl.BlockSpec((1, H), lambda i: (0, 0)),
            pl.BlockSpec((1, 1), lambda i: (0, 0)),
            pl.BlockSpec((1, H), lambda i: (0, 0)),
        ],
        out_specs=pl.BlockSpec((be, ROW), lambda i: (i, 0)),
        out_shape=jax.ShapeDtypeStruct((e, ROW), F32),
    )(sarr, ea_t, we, emw, emb, cmw, cmb, wr)


def _tc_node0(x, pos, acc0, acc1, w1x, w1e, b1, w2, b2, wi1, wj1, bn):
    n, f = x.shape

    def body(x_ref, pos_ref, a0_ref, a1_ref, w1x_ref, w1e_ref, b1_ref,
             w2_ref, b2_ref, wi_ref, wj_ref, x1_ref, ti_ref, tj_ref):
        a0 = a0_ref[...]
        a1 = a1_ref[...]
        e_agg = a0[:, :H] + a1[:, :H]
        csum = a0[:, H:H + 3] + a1[:, H:H + 3]
        cnt = a0[:, H + 3:H + 4] + a1[:, H + 3:H + 4]
        posn = pos_ref[...] + csum / jnp.maximum(cnt, 1.0)
        h = _silu(jnp.dot(x_ref[...], w1x_ref[...], preferred_element_type=F32)
                  + jnp.dot(e_agg, w1e_ref[...], preferred_element_type=F32)
                  + b1_ref[...])
        x1 = jnp.dot(h, w2_ref[...], preferred_element_type=F32) + b2_ref[...]
        x1_ref[...] = x1
        pad = jnp.zeros((bn, ROW - H - 3), F32)
        ti_ref[...] = jnp.concatenate(
            [jnp.dot(x1, wi_ref[...], preferred_element_type=F32), posn, pad], 1)
        tj_ref[...] = jnp.concatenate(
            [jnp.dot(x1, wj_ref[...], preferred_element_type=F32), -posn, pad], 1)

    c0 = pl.BlockSpec((H, H), lambda i: (0, 0))
    cb = pl.BlockSpec((1, H), lambda i: (0, 0))
    return pl.pallas_call(
        body,
        grid=(n // bn,),
        in_specs=[
            pl.BlockSpec((bn, f), lambda i: (i, 0)),
            pl.BlockSpec((bn, 3), lambda i: (i, 0)),
            pl.BlockSpec((bn, AW), lambda i: (i, 0)),
            pl.BlockSpec((bn, AW), lambda i: (i, 0)),
            pl.BlockSpec((f, H), lambda i: (0, 0)),
            c0, cb, c0, cb, c0, c0,
        ],
        out_specs=[
            pl.BlockSpec((bn, H), lambda i: (i, 0)),
            pl.BlockSpec((bn, ROW), lambda i: (i, 0)),
            pl.BlockSpec((bn, ROW), lambda i: (i, 0)),
        ],
        out_shape=[
            jax.ShapeDtypeStruct((n, H), F32),
            jax.ShapeDtypeStruct((n, ROW), F32),
            jax.ShapeDtypeStruct((n, ROW), F32),
        ],
    )(x, pos, acc0, acc1, w1x, w1e, b1, w2, b2, wi1, wj1)


def _tc_node1_pool(x1, acc0, acc1, batch_col, w1x, w1e, b1, w2, b2, bn):
    n = x1.shape[0]
    g = 64

    def body(x_ref, a0_ref, a1_ref, b_ref, w1x_ref, w1e_ref, b1_ref,
             w2_ref, b2_ref, out_ref):
        e_agg = a0_ref[:, :H] + a1_ref[:, :H]
        h = _silu(jnp.dot(x_ref[...], w1x_ref[...], preferred_element_type=F32)
                  + jnp.dot(e_agg, w1e_ref[...], preferred_element_type=F32)
                  + b1_ref[...])
        x2 = jnp.dot(h, w2_ref[...], preferred_element_type=F32) + b2_ref[...]
        gid = lax.broadcasted_iota(jnp.int32, (1, g), 1)
        oh = (b_ref[...] == gid).astype(F32)
        feat = jnp.concatenate(
            [x2, jnp.ones((bn, 1), F32), jnp.zeros((bn, 7), F32)], 1)
        contrib = lax.dot_general(oh, feat, (((0,), (0,)), ((), ())),
                                  preferred_element_type=F32)

        @pl.when(pl.program_id(0) == 0)
        def _():
            out_ref[...] = contrib

        @pl.when(pl.program_id(0) != 0)
        def _():
            out_ref[...] = out_ref[...] + contrib

    c0 = pl.BlockSpec((H, H), lambda i: (0, 0))
    cb = pl.BlockSpec((1, H), lambda i: (0, 0))
    return pl.pallas_call(
        body,
        grid=(n // bn,),
        in_specs=[
            pl.BlockSpec((bn, H), lambda i: (i, 0)),
            pl.BlockSpec((bn, AW), lambda i: (i, 0)),
            pl.BlockSpec((bn, AW), lambda i: (i, 0)),
            pl.BlockSpec((bn, 1), lambda i: (i, 0)),
            c0, c0, cb, c0, cb,
        ],
        out_specs=pl.BlockSpec((g, H + 8), lambda i: (0, 0)),
        out_shape=jax.ShapeDtypeStruct((g, H + 8), F32),
    )(x1, acc0, acc1, batch_col, w1x, w1e, b1, w2, b2)


def _tc_final(pooled, w1, b1, w2, b2):
    g = pooled.shape[0]
    out_c = w2.shape[1]

    def body(p_ref, w1_ref, b1_ref, w2_ref, b2_ref, o_ref):
        p = p_ref[...]
        xg = p[:, :H] / jnp.maximum(p[:, H:H + 1], 1.0)
        m = jnp.max(xg, axis=1, keepdims=True)
        z = xg - m
        xl = z - jnp.log(jnp.sum(jnp.exp(z), axis=1, keepdims=True))
        hh = jnp.maximum(
            jnp.dot(xl, w1_ref[...], preferred_element_type=F32) + b1_ref[...],
            0.0)
        o_ref[...] = (jnp.dot(hh, w2_ref[...], preferred_element_type=F32)
                      + b2_ref[...])

    return pl.pallas_call(
        body,
        out_shape=jax.ShapeDtypeStruct((g, out_c), F32),
    )(pooled, w1, b1, w2, b2)


def kernel(x, pos, edge_attr, edge_index, batch, params):
    n, f = x.shape
    d = edge_attr.shape[1]
    src = edge_index[0]
    dst = edge_index[1]

    p0, p1, po = params['l0'], params['l1'], params['out']
    w0 = p0['mlp_W']
    wi0, wj0 = w0[:f], w0[f:2 * f]
    we0, wr0 = w0[2 * f:2 * f + d], w0[2 * f + d].reshape(1, H)
    w1 = p1['mlp_W']
    wi1, wj1 = w1[:H], w1[H:2 * H]
    we1, wr1 = w1[2 * H:2 * H + d], w1[2 * H + d].reshape(1, H)

    zrows = jnp.zeros((80, AW), F32)
    batch_col = batch.reshape(n, 1)
    ea_t = edge_attr.T

    ti0, tj0 = _tc_prep_tables(x, pos, wi0, wj0, bn=2000)

    s0 = _sc_gather_sum(ti0, tj0, src, dst)
    u0 = _tc_edge_mlp(s0, ea_t, we0, p0['em_W'], p0['em_b'].reshape(1, H),
                      p0['cm_W'].reshape(1, H), p0['cm_b'].reshape(1, 1),
                      wr0, be=2560, with_coord=True)
    acc = _sc_scatter_add(u0, dst, zrows)
    x1, ti1, tj1 = _tc_node0(
        x, pos, acc[0], acc[1],
        p0['nm_W1'][:f], p0['nm_W1'][f:], p0['nm_b1'].reshape(1, H),
        p0['nm_W2'], p0['nm_b2'].reshape(1, H), wi1, wj1, bn=2000)

    s1 = _sc_gather_sum(ti1, tj1, src, dst)
    u1 = _tc_edge_mlp(s1, ea_t, we1, p1['em_W'], p1['em_b'].reshape(1, H),
                      p1['cm_W'].reshape(1, H), p1['cm_b'].reshape(1, 1),
                      wr1, be=2560, with_coord=False)
    acc1 = _sc_scatter_add(u1, dst, zrows)
    pooled = _tc_node1_pool(
        x1, acc1[0], acc1[1], batch_col,
        p1['nm_W1'][:H], p1['nm_W1'][H:], p1['nm_b1'].reshape(1, H),
        p1['nm_W2'], p1['nm_b2'].reshape(1, H), bn=2000)

    return _tc_final(pooled, po['W1'], po['b1'].reshape(1, H),
                     po['W2'], po['b2'].reshape(1, po['W2'].shape[1]))


# speedup vs baseline: 7.8899x; 1.2310x over previous
import functools

import jax
import jax.numpy as jnp
from jax import lax
from jax.experimental import pallas as pl
from jax.experimental.pallas import tpu as pltpu
from jax.experimental.pallas import tpu_sc as plsc

F32 = jnp.float32
ROW = 128
AW = 72
H = 64


def _silu(v):
    return v * (1.0 / (1.0 + jnp.exp(-v)))


def _sc_gather_sum(ti, tj, src, dst, eoff, ecnt):
    e = ecnt
    BLK = 64
    nblk = e // BLK
    maxp = (nblk // 32 + 3) // 3
    mesh = plsc.VectorSubcoreMesh(core_axis_name="c", subcore_axis_name="s")

    @functools.partial(
        pl.kernel,
        out_type=jax.ShapeDtypeStruct((e, ROW), F32),
        mesh=mesh,
        scratch_types=[
            [pltpu.VMEM((BLK,), jnp.int32)] * 3,
            [pltpu.VMEM((BLK,), jnp.int32)] * 3,
            [pltpu.VMEM((BLK, ROW), F32)] * 3,
            [pltpu.VMEM((BLK, ROW), F32)] * 3,
            [pltpu.VMEM((BLK, ROW), F32)] * 3,
            [pltpu.SemaphoreType.DMA] * 3,
            [pltpu.SemaphoreType.DMA] * 3,
            [pltpu.SemaphoreType.DMA] * 3,
            [pltpu.SemaphoreType.DMA] * 3,
            [pltpu.SemaphoreType.DMA] * 3,
        ],
        compiler_params=pltpu.CompilerParams(use_tc_tiling_on_sc=False),
    )
    def k(ti_hbm, tj_hbm, src_hbm, dst_hbm, out_hbm,
          idd, ids, ri, rj, ro, sid, sis, sgi, sgj, sw):
        c = lax.axis_index("c")
        s = lax.axis_index("s")
        wid = s * 2 + c
        nb = (nblk - wid + 31) // 32

        def start_idx(q, it):
            @pl.when(it < nb)
            def _():
                e0 = eoff + (wid + it * 32) * BLK
                pltpu.make_async_copy(dst_hbm.at[pl.ds(e0, BLK)], idd[q],
                                      sid[q]).start()
                pltpu.make_async_copy(src_hbm.at[pl.ds(e0, BLK)], ids[q],
                                      sis[q]).start()

        def start_gather(q, it):
            @pl.when(it < nb)
            def _():
                pltpu.make_async_copy(dst_hbm.at[pl.ds(0, BLK)], idd[q],
                                      sid[q]).wait()
                pltpu.make_async_copy(src_hbm.at[pl.ds(0, BLK)], ids[q],
                                      sis[q]).wait()
                pltpu.make_async_copy(ti_hbm.at[idd[q]], ri[q], sgi[q]).start()
                pltpu.make_async_copy(tj_hbm.at[ids[q]], rj[q], sgj[q]).start()

        def finish(q, it):
            @pl.when(it < nb)
            def _():
                e0 = (wid + it * 32) * BLK
                pltpu.make_async_copy(ti_hbm.at[pl.ds(0, BLK)], ri[q],
                                      sgi[q]).wait()
                pltpu.make_async_copy(tj_hbm.at[pl.ds(0, BLK)], rj[q],
                                      sgj[q]).wait()

                @pl.when(it >= 3)
                def _():
                    pltpu.make_async_copy(ro[q], out_hbm.at[pl.ds(0, BLK)],
                                          sw[q]).wait()

                def row(r, carry):
                    for jj in range(5):
                        sl = pl.ds(jj * 16, 16)
                        ro[q][r, sl] = ri[q][r, sl] + rj[q][r, sl]
                    return carry

                lax.fori_loop(0, BLK, row, 0, unroll=4)
                pltpu.make_async_copy(ro[q], out_hbm.at[pl.ds(e0, BLK)],
                                      sw[q]).start()

        start_idx(0, 0)
        start_idx(1, 1)
        start_idx(2, 2)
        start_gather(0, 0)
        start_gather(1, 1)

        def body(p, carry):
            for q in range(3):
                it = 3 * p + q
                finish(q, it)
                start_idx(q, it + 3)
                start_gather((q + 2) % 3, it + 2)
            return carry

        lax.fori_loop(0, maxp, body, 0)
        for q in range(3):
            pltpu.make_async_copy(ro[q], out_hbm.at[pl.ds(0, BLK)],
                                  sw[q]).wait()

    return k(ti, tj, src, dst)


def _sc_scatter_add(u, dst, zrows, eoff):
    e = u.shape[0]
    n = 10000
    nblk = e // 128
    half = nblk // 2
    maxp = (half // 16 + 2) // 2
    zr = zrows.shape[0]
    mesh = plsc.VectorSubcoreMesh(core_axis_name="c", subcore_axis_name="s")

    @functools.partial(
        pl.kernel,
        out_type=jax.ShapeDtypeStruct((2, n, AW), F32),
        mesh=mesh,
        scratch_types=[
            pltpu.VMEM((128,), jnp.int32), pltpu.VMEM((128,), jnp.int32),
            pltpu.VMEM((128, AW), F32), pltpu.VMEM((128, AW), F32),
            pltpu.VMEM((zr, AW), F32),
            pltpu.VMEM_SHARED((n, AW), F32),
            pltpu.SemaphoreType.DMA, pltpu.SemaphoreType.DMA,
            pltpu.SemaphoreType.DMA, pltpu.SemaphoreType.DMA,
            pltpu.SemaphoreType.DMA, pltpu.SemaphoreType.DMA,
        ],
        compiler_params=pltpu.CompilerParams(use_tc_tiling_on_sc=False),
    )
    def k(u_hbm, dst_hbm, z_hbm, out_hbm, ix0, ix1, u0, u1, z_v, acc,
          li0, lu0, li1, lu1, as0, as1):
        c = lax.axis_index("c")
        s = lax.axis_index("s")
        pltpu.sync_copy(z_hbm, z_v)
        nchunk = jnp.where(s == 15, 5, 8)

        def zbody(kk, carry):
            pltpu.sync_copy(z_v, acc.at[pl.ds(s * 640 + kk * zr, zr)])
            return carry

        lax.fori_loop(0, nchunk, zbody, 0)
        plsc.subcore_barrier()

        nb = (half - s + 15) // 16

        def start(it, ix, uv, li, lu, asem):
            @pl.when(jnp.logical_and(it < nb, it >= 2))
            def _():
                pltpu.make_async_copy(uv, acc.at[pl.ds(0, 128)], asem).wait()

            @pl.when(it < nb)
            def _():
                e0 = (c * half + s + it * 16) * 128
                pltpu.make_async_copy(dst_hbm.at[pl.ds(eoff + e0, 128)], ix,
                                      li).start()
                pltpu.make_async_copy(
                    u_hbm.at[pl.ds(e0, 128), pl.ds(0, AW)], uv, lu).start()

        def process(it, ix, uv, li, lu, asem):
            @pl.when(it < nb)
            def _():
                pltpu.make_async_copy(dst_hbm.at[pl.ds(0, 128)], ix,
                                      li).wait()
                pltpu.make_async_copy(u_hbm.at[pl.ds(0, 128), pl.ds(0, AW)],
                                      uv, lu).wait()
                pltpu.make_async_copy(uv, acc.at[ix], asem).start(add=True)

        start(0, ix0, u0, li0, lu0, as0)
        start(1, ix1, u1, li1, lu1, as1)

        def body(p, carry):
            it = 2 * p
            process(it, ix0, u0, li0, lu0, as0)
            start(it + 2, ix0, u0, li0, lu0, as0)
            process(it + 1, ix1, u1, li1, lu1, as1)
            start(it + 3, ix1, u1, li1, lu1, as1)
            return carry

        lax.fori_loop(0, maxp, body, 0)
        pltpu.make_async_copy(u0, acc.at[pl.ds(0, 128)], as0).wait()
        pltpu.make_async_copy(u1, acc.at[pl.ds(0, 128)], as1).wait()
        plsc.subcore_barrier()

        def wbody(kk, carry):
            r0 = s * 640 + kk * zr
            pltpu.sync_copy(acc.at[pl.ds(r0, zr)], z_v)
            pltpu.sync_copy(z_v, out_hbm.at[c, pl.ds(r0, zr)])
            return carry

        lax.fori_loop(0, nchunk, wbody, 0)

    return k(u, dst, zrows)


def _tc_prep_tables(x, pos, wi, wj, bn):
    n, f = x.shape

    def body(x_ref, pos_ref, wi_ref, wj_ref, ti_ref, tj_ref):
        xb = x_ref[...]
        p = pos_ref[...]
        pad = jnp.zeros((bn, ROW - H - 3), F32)
        ti_ref[...] = jnp.concatenate(
            [jnp.dot(xb, wi_ref[...], preferred_element_type=F32), p, pad], 1)
        tj_ref[...] = jnp.concatenate(
            [jnp.dot(xb, wj_ref[...], preferred_element_type=F32), -p, pad], 1)

    return pl.pallas_call(
        body,
        grid=(n // bn,),
        in_specs=[
            pl.BlockSpec((bn, f), lambda i: (i, 0)),
            pl.BlockSpec((bn, 3), lambda i: (i, 0)),
            pl.BlockSpec((f, H), lambda i: (0, 0)),
            pl.BlockSpec((f, H), lambda i: (0, 0)),
        ],
        out_specs=[
            pl.BlockSpec((bn, ROW), lambda i: (i, 0)),
            pl.BlockSpec((bn, ROW), lambda i: (i, 0)),
        ],
        out_shape=[
            jax.ShapeDtypeStruct((n, ROW), F32),
            jax.ShapeDtypeStruct((n, ROW), F32),
        ],
    )(x, pos, wi, wj)


def _tc_edge_mlp(sarr, ea_t, we, emw, emb, cmw, cmb, wr, be, with_coord,
                 blk_off=0):
    e = sarr.shape[0]
    d = ea_t.shape[0]

    def body(s_ref, ea_ref, we_ref, emw_ref, emb_ref, cmw_ref, cmb_ref,
             wr_ref, u_ref):
        sb = s_ref[...]
        xpart = sb[:, :H]
        diff = sb[:, H:H + 3]
        radial = jnp.sum(diff * diff, axis=1, keepdims=True)
        pre = (xpart
               + lax.dot_general(ea_ref[...], we_ref[...],
                                 (((0,), (0,)), ((), ())),
                                 preferred_element_type=F32)
               + radial * wr_ref[...])
        er = _silu(pre)
        eij = _silu(jnp.dot(er, emw_ref[...], preferred_element_type=F32)
                    + emb_ref[...])
        if with_coord:
            sc = _silu(jnp.sum(eij * cmw_ref[...], axis=1, keepdims=True)
                       + cmb_ref[...])
            u_ref[...] = jnp.concatenate(
                [eij, diff * sc, jnp.ones((be, 1), F32),
                 jnp.zeros((be, ROW - H - 4), F32)], 1)
        else:
            u_ref[...] = jnp.concatenate([eij, eij], 1)

    ob = blk_off
    return pl.pallas_call(
        body,
        grid=(e // be,),
        in_specs=[
            pl.BlockSpec((be, ROW), lambda i: (i, 0)),
            pl.BlockSpec((d, be), lambda i: (0, i + ob)),
            pl.BlockSpec((d, H), lambda i: (0, 0)),
            pl.BlockSpec((H, H), lambda i: (0, 0)),
            pl.BlockSpec((1, H), lambda i: (0, 0)),
            pl.BlockSpec((1, H), lambda i: (0, 0)),
            pl.BlockSpec((1, 1), lambda i: (0, 0)),
            pl.BlockSpec((1, H), lambda i: (0, 0)),
        ],
        out_specs=pl.BlockSpec((be, ROW), lambda i: (i, 0)),
        out_shape=jax.ShapeDtypeStruct((e, ROW), F32),
    )(sarr, ea_t, we, emw, emb, cmw, cmb, wr)


def _tc_node0(x, pos, accs, w1x, w1e, b1, w2, b2, wi1, wj1, bn):
    n, f = x.shape

    def body(x_ref, pos_ref, a0_ref, a1_ref, a2_ref, a3_ref, w1x_ref,
             w1e_ref, b1_ref, w2_ref, b2_ref, wi_ref, wj_ref,
             x1_ref, ti_ref, tj_ref):
        a0 = a0_ref[...] + a1_ref[...] + a2_ref[...] + a3_ref[...]
        e_agg = a0[:, :H]
        csum = a0[:, H:H + 3]
        cnt = a0[:, H + 3:H + 4]
        posn = pos_ref[...] + csum / jnp.maximum(cnt, 1.0)
        h = _silu(jnp.dot(x_ref[...], w1x_ref[...], preferred_element_type=F32)
                  + jnp.dot(e_agg, w1e_ref[...], preferred_element_type=F32)
                  + b1_ref[...])
        x1 = jnp.dot(h, w2_ref[...], preferred_element_type=F32) + b2_ref[...]
        x1_ref[...] = x1
        pad = jnp.zeros((bn, ROW - H - 3), F32)
        ti_ref[...] = jnp.concatenate(
            [jnp.dot(x1, wi_ref[...], preferred_element_type=F32), posn, pad], 1)
        tj_ref[...] = jnp.concatenate(
            [jnp.dot(x1, wj_ref[...], preferred_element_type=F32), -posn, pad], 1)

    c0 = pl.BlockSpec((H, H), lambda i: (0, 0))
    cb = pl.BlockSpec((1, H), lambda i: (0, 0))
    return pl.pallas_call(
        body,
        grid=(n // bn,),
        in_specs=[
            pl.BlockSpec((bn, f), lambda i: (i, 0)),
            pl.BlockSpec((bn, 3), lambda i: (i, 0)),
            pl.BlockSpec((bn, AW), lambda i: (i, 0)),
            pl.BlockSpec((bn, AW), lambda i: (i, 0)),
            pl.BlockSpec((bn, AW), lambda i: (i, 0)),
            pl.BlockSpec((bn, AW), lambda i: (i, 0)),
            pl.BlockSpec((f, H), lambda i: (0, 0)),
            c0, cb, c0, cb, c0, c0,
        ],
        out_specs=[
            pl.BlockSpec((bn, H), lambda i: (i, 0)),
            pl.BlockSpec((bn, ROW), lambda i: (i, 0)),
            pl.BlockSpec((bn, ROW), lambda i: (i, 0)),
        ],
        out_shape=[
            jax.ShapeDtypeStruct((n, H), F32),
            jax.ShapeDtypeStruct((n, ROW), F32),
            jax.ShapeDtypeStruct((n, ROW), F32),
        ],
    )(x, pos, *accs, w1x, w1e, b1, w2, b2, wi1, wj1)


def _tc_node1_pool(x1, accs, batch_col, w1x, w1e, b1, w2, b2, bn):
    n = x1.shape[0]
    g = 64

    def body(x_ref, a0_ref, a1_ref, a2_ref, a3_ref, b_ref, w1x_ref,
             w1e_ref, b1_ref, w2_ref, b2_ref, out_ref):
        e_agg = (a0_ref[:, :H] + a1_ref[:, :H]
                 + a2_ref[:, :H] + a3_ref[:, :H])
        h = _silu(jnp.dot(x_ref[...], w1x_ref[...], preferred_element_type=F32)
                  + jnp.dot(e_agg, w1e_ref[...], preferred_element_type=F32)
                  + b1_ref[...])
        x2 = jnp.dot(h, w2_ref[...], preferred_element_type=F32) + b2_ref[...]
        gid = lax.broadcasted_iota(jnp.int32, (1, g), 1)
        oh = (b_ref[...] == gid).astype(F32)
        feat = jnp.concatenate(
            [x2, jnp.ones((bn, 1), F32), jnp.zeros((bn, 7), F32)], 1)
        contrib = lax.dot_general(oh, feat, (((0,), (0,)), ((), ())),
                                  preferred_element_type=F32)

        @pl.when(pl.program_id(0) == 0)
        def _():
            out_ref[...] = contrib

        @pl.when(pl.program_id(0) != 0)
        def _():
            out_ref[...] = out_ref[...] + contrib

    c0 = pl.BlockSpec((H, H), lambda i: (0, 0))
    cb = pl.BlockSpec((1, H), lambda i: (0, 0))
    return pl.pallas_call(
        body,
        grid=(n // bn,),
        in_specs=[
            pl.BlockSpec((bn, H), lambda i: (i, 0)),
            pl.BlockSpec((bn, AW), lambda i: (i, 0)),
            pl.BlockSpec((bn, AW), lambda i: (i, 0)),
            pl.BlockSpec((bn, AW), lambda i: (i, 0)),
            pl.BlockSpec((bn, AW), lambda i: (i, 0)),
            pl.BlockSpec((bn, 1), lambda i: (i, 0)),
            c0, c0, cb, c0, cb,
        ],
        out_specs=pl.BlockSpec((g, H + 8), lambda i: (0, 0)),
        out_shape=jax.ShapeDtypeStruct((g, H + 8), F32),
    )(x1, *accs, batch_col, w1x, w1e, b1, w2, b2)


def _tc_final(pooled, w1, b1, w2, b2):
    g = pooled.shape[0]
    out_c = w2.shape[1]

    def body(p_ref, w1_ref, b1_ref, w2_ref, b2_ref, o_ref):
        p = p_ref[...]
        xg = p[:, :H] / jnp.maximum(p[:, H:H + 1], 1.0)
        m = jnp.max(xg, axis=1, keepdims=True)
        z = xg - m
        xl = z - jnp.log(jnp.sum(jnp.exp(z), axis=1, keepdims=True))
        hh = jnp.maximum(
            jnp.dot(xl, w1_ref[...], preferred_element_type=F32) + b1_ref[...],
            0.0)
        o_ref[...] = (jnp.dot(hh, w2_ref[...], preferred_element_type=F32)
                      + b2_ref[...])

    return pl.pallas_call(
        body,
        out_shape=jax.ShapeDtypeStruct((g, out_c), F32),
    )(pooled, w1, b1, w2, b2)


def kernel(x, pos, edge_attr, edge_index, batch, params):
    n, f = x.shape
    d = edge_attr.shape[1]
    src = edge_index[0]
    dst = edge_index[1]

    p0, p1, po = params['l0'], params['l1'], params['out']
    w0 = p0['mlp_W']
    wi0, wj0 = w0[:f], w0[f:2 * f]
    we0, wr0 = w0[2 * f:2 * f + d], w0[2 * f + d].reshape(1, H)
    w1 = p1['mlp_W']
    wi1, wj1 = w1[:H], w1[H:2 * H]
    we1, wr1 = w1[2 * H:2 * H + d], w1[2 * H + d].reshape(1, H)

    zrows = jnp.zeros((80, AW), F32)
    batch_col = batch.reshape(n, 1)
    ea_t = edge_attr.T

    ti0, tj0 = _tc_prep_tables(x, pos, wi0, wj0, bn=2000)

    e = src.shape[0]
    eh = e // 2
    bh = eh // 3200

    def run_layer(ti, tj, we, wr, p, with_coord):
        accs = []
        for ci in range(2):
            sarr = _sc_gather_sum(ti, tj, src, dst, ci * eh, eh)
            u = _tc_edge_mlp(sarr, ea_t, we, p['em_W'],
                             p['em_b'].reshape(1, H), p['cm_W'].reshape(1, H),
                             p['cm_b'].reshape(1, 1), wr, be=3200,
                             with_coord=with_coord, blk_off=ci * bh)
            a = _sc_scatter_add(u, dst, zrows, ci * eh)
            accs.extend([a[0], a[1]])
        return accs

    accs0 = run_layer(ti0, tj0, we0, wr0, p0, True)
    x1, ti1, tj1 = _tc_node0(
        x, pos, accs0,
        p0['nm_W1'][:f], p0['nm_W1'][f:], p0['nm_b1'].reshape(1, H),
        p0['nm_W2'], p0['nm_b2'].reshape(1, H), wi1, wj1, bn=2000)

    accs1 = run_layer(ti1, tj1, we1, wr1, p1, False)
    pooled = _tc_node1_pool(
        x1, accs1, batch_col,
        p1['nm_W1'][:H], p1['nm_W1'][H:], p1['nm_b1'].reshape(1, H),
        p1['nm_W2'], p1['nm_b2'].reshape(1, H), bn=2000)

    return _tc_final(pooled, po['W1'], po['b1'].reshape(1, H),
                     po['W2'], po['b2'].reshape(1, po['W2'].shape[1]))


# speedup vs baseline: 8.1181x; 1.0289x over previous
import functools

import jax
import jax.numpy as jnp
from jax import lax
from jax.experimental import pallas as pl
from jax.experimental.pallas import tpu as pltpu
from jax.experimental.pallas import tpu_sc as plsc

F32 = jnp.float32
ROW = 128
AW = 72
H = 64


def _silu(v):
    return v * (1.0 / (1.0 + jnp.exp(-v)))


def _sc_gather_sum(ti, tj, src, dst, eoff, ecnt):
    e = ecnt
    BLK = 64
    nblk = e // BLK
    maxp = (nblk // 32 + 3) // 3
    mesh = plsc.VectorSubcoreMesh(core_axis_name="c", subcore_axis_name="s")

    @functools.partial(
        pl.kernel,
        out_type=jax.ShapeDtypeStruct((e, ROW), F32),
        mesh=mesh,
        scratch_types=[
            [pltpu.VMEM((BLK,), jnp.int32)] * 3,
            [pltpu.VMEM((BLK,), jnp.int32)] * 3,
            [pltpu.VMEM((BLK, ROW), F32)] * 3,
            [pltpu.VMEM((BLK, ROW), F32)] * 3,
            [pltpu.VMEM((BLK, ROW), F32)] * 3,
            [pltpu.SemaphoreType.DMA] * 3,
            [pltpu.SemaphoreType.DMA] * 3,
            [pltpu.SemaphoreType.DMA] * 3,
            [pltpu.SemaphoreType.DMA] * 3,
            [pltpu.SemaphoreType.DMA] * 3,
        ],
        compiler_params=pltpu.CompilerParams(use_tc_tiling_on_sc=False),
    )
    def k(ti_hbm, tj_hbm, src_hbm, dst_hbm, out_hbm,
          idd, ids, ri, rj, ro, sid, sis, sgi, sgj, sw):
        c = lax.axis_index("c")
        s = lax.axis_index("s")
        wid = s * 2 + c
        nb = (nblk - wid + 31) // 32

        def start_idx(q, it):
            @pl.when(it < nb)
            def _():
                e0 = eoff + (wid + it * 32) * BLK
                pltpu.make_async_copy(dst_hbm.at[pl.ds(e0, BLK)], idd[q],
                                      sid[q]).start()
                pltpu.make_async_copy(src_hbm.at[pl.ds(e0, BLK)], ids[q],
                                      sis[q]).start()

        def start_gather(q, it):
            @pl.when(it < nb)
            def _():
                pltpu.make_async_copy(dst_hbm.at[pl.ds(0, BLK)], idd[q],
                                      sid[q]).wait()
                pltpu.make_async_copy(src_hbm.at[pl.ds(0, BLK)], ids[q],
                                      sis[q]).wait()
                pltpu.make_async_copy(ti_hbm.at[idd[q]], ri[q], sgi[q]).start()
                pltpu.make_async_copy(tj_hbm.at[ids[q]], rj[q], sgj[q]).start()

        def finish(q, it):
            @pl.when(it < nb)
            def _():
                e0 = (wid + it * 32) * BLK
                pltpu.make_async_copy(ti_hbm.at[pl.ds(0, BLK)], ri[q],
                                      sgi[q]).wait()
                pltpu.make_async_copy(tj_hbm.at[pl.ds(0, BLK)], rj[q],
                                      sgj[q]).wait()

                @pl.when(it >= 3)
                def _():
                    pltpu.make_async_copy(ro[q], out_hbm.at[pl.ds(0, BLK)],
                                          sw[q]).wait()

                def row(r, carry):
                    for jj in range(5):
                        sl = pl.ds(jj * 16, 16)
                        ro[q][r, sl] = ri[q][r, sl] + rj[q][r, sl]
                    return carry

                lax.fori_loop(0, BLK, row, 0, unroll=4)
                pltpu.make_async_copy(ro[q], out_hbm.at[pl.ds(e0, BLK)],
                                      sw[q]).start()

        def zrow(r, carry):
            zv = jnp.zeros((16,), F32)
            for q in range(3):
                for off in (80, 96, 112):
                    ro[q][r, pl.ds(off, 16)] = zv
            return carry

        lax.fori_loop(0, BLK, zrow, 0, unroll=4)

        start_idx(0, 0)
        start_idx(1, 1)
        start_idx(2, 2)
        start_gather(0, 0)
        start_gather(1, 1)

        def body(p, carry):
            for q in range(3):
                it = 3 * p + q
                finish(q, it)
                start_idx(q, it + 3)
                start_gather((q + 2) % 3, it + 2)
            return carry

        lax.fori_loop(0, maxp, body, 0)
        for q in range(3):
            pltpu.make_async_copy(ro[q], out_hbm.at[pl.ds(0, BLK)],
                                  sw[q]).wait()

    return k(ti, tj, src, dst)


def _sc_scatter_add(u, dst, zrows, eoff):
    e = u.shape[0]
    n = 10000
    nblk = e // 128
    half = nblk // 2
    maxp = (half // 16 + 2) // 2
    zr = zrows.shape[0]
    mesh = plsc.VectorSubcoreMesh(core_axis_name="c", subcore_axis_name="s")

    @functools.partial(
        pl.kernel,
        out_type=jax.ShapeDtypeStruct((2, n, AW), F32),
        mesh=mesh,
        scratch_types=[
            pltpu.VMEM((128,), jnp.int32), pltpu.VMEM((128,), jnp.int32),
            pltpu.VMEM((128, AW), F32), pltpu.VMEM((128, AW), F32),
            pltpu.VMEM((zr, AW), F32),
            pltpu.VMEM_SHARED((n, AW), F32),
            pltpu.SemaphoreType.DMA, pltpu.SemaphoreType.DMA,
            pltpu.SemaphoreType.DMA, pltpu.SemaphoreType.DMA,
            pltpu.SemaphoreType.DMA, pltpu.SemaphoreType.DMA,
        ],
        compiler_params=pltpu.CompilerParams(use_tc_tiling_on_sc=False),
    )
    def k(u_hbm, dst_hbm, z_hbm, out_hbm, ix0, ix1, u0, u1, z_v, acc,
          li0, lu0, li1, lu1, as0, as1):
        c = lax.axis_index("c")
        s = lax.axis_index("s")
        pltpu.sync_copy(z_hbm, z_v)
        nchunk = jnp.where(s == 15, 5, 8)

        def zbody(kk, carry):
            pltpu.sync_copy(z_v, acc.at[pl.ds(s * 640 + kk * zr, zr)])
            return carry

        lax.fori_loop(0, nchunk, zbody, 0)
        plsc.subcore_barrier()

        nb = (half - s + 15) // 16

        def start(it, ix, uv, li, lu, asem):
            @pl.when(jnp.logical_and(it < nb, it >= 2))
            def _():
                pltpu.make_async_copy(uv, acc.at[pl.ds(0, 128)], asem).wait()

            @pl.when(it < nb)
            def _():
                e0 = (c * half + s + it * 16) * 128
                pltpu.make_async_copy(dst_hbm.at[pl.ds(eoff + e0, 128)], ix,
                                      li).start()
                pltpu.make_async_copy(
                    u_hbm.at[pl.ds(e0, 128), pl.ds(0, AW)], uv, lu).start()

        def process(it, ix, uv, li, lu, asem):
            @pl.when(it < nb)
            def _():
                pltpu.make_async_copy(dst_hbm.at[pl.ds(0, 128)], ix,
                                      li).wait()
                pltpu.make_async_copy(u_hbm.at[pl.ds(0, 128), pl.ds(0, AW)],
                                      uv, lu).wait()
                pltpu.make_async_copy(uv, acc.at[ix], asem).start(add=True)

        start(0, ix0, u0, li0, lu0, as0)
        start(1, ix1, u1, li1, lu1, as1)

        def body(p, carry):
            it = 2 * p
            process(it, ix0, u0, li0, lu0, as0)
            start(it + 2, ix0, u0, li0, lu0, as0)
            process(it + 1, ix1, u1, li1, lu1, as1)
            start(it + 3, ix1, u1, li1, lu1, as1)
            return carry

        lax.fori_loop(0, maxp, body, 0)
        pltpu.make_async_copy(u0, acc.at[pl.ds(0, 128)], as0).wait()
        pltpu.make_async_copy(u1, acc.at[pl.ds(0, 128)], as1).wait()
        plsc.subcore_barrier()

        def wbody(kk, carry):
            r0 = s * 640 + kk * zr
            pltpu.sync_copy(acc.at[pl.ds(r0, zr)], z_v)
            pltpu.sync_copy(z_v, out_hbm.at[c, pl.ds(r0, zr)])
            return carry

        lax.fori_loop(0, nchunk, wbody, 0)

    return k(u, dst, zrows)


def _tc_prep_tables(x, pos, wi, wj, bn):
    n, f = x.shape

    def body(x_ref, pos_ref, wi_ref, wj_ref, ti_ref, tj_ref):
        xb = x_ref[...]
        p = pos_ref[...]
        pad = jnp.zeros((bn, ROW - H - 3), F32)
        ti_ref[...] = jnp.concatenate(
            [jnp.dot(xb, wi_ref[...], preferred_element_type=F32), p, pad], 1)
        tj_ref[...] = jnp.concatenate(
            [jnp.dot(xb, wj_ref[...], preferred_element_type=F32), -p, pad], 1)

    return pl.pallas_call(
        body,
        grid=(n // bn,),
        in_specs=[
            pl.BlockSpec((bn, f), lambda i: (i, 0)),
            pl.BlockSpec((bn, 3), lambda i: (i, 0)),
            pl.BlockSpec((f, H), lambda i: (0, 0)),
            pl.BlockSpec((f, H), lambda i: (0, 0)),
        ],
        out_specs=[
            pl.BlockSpec((bn, ROW), lambda i: (i, 0)),
            pl.BlockSpec((bn, ROW), lambda i: (i, 0)),
        ],
        out_shape=[
            jax.ShapeDtypeStruct((n, ROW), F32),
            jax.ShapeDtypeStruct((n, ROW), F32),
        ],
    )(x, pos, wi, wj)


def _tc_edge_mlp(sarr, ea_t, we, mr, emw, emb, cmw8, cmb, konst, be,
                 with_coord, blk_off=0):
    e = sarr.shape[0]
    d = ea_t.shape[0]

    def body(s_ref, ea_ref, we_ref, mr_ref, emw_ref, emb_ref, cmw_ref,
             cmb_ref, k_ref, u_ref):
        sb = s_ref[...]
        pre = (sb[:, :H]
               + lax.dot_general(ea_ref[...], we_ref[...],
                                 (((0,), (0,)), ((), ())),
                                 preferred_element_type=F32)
               + jnp.dot(sb * sb, mr_ref[...], preferred_element_type=F32))
        er = _silu(pre)
        eij = _silu(jnp.dot(er, emw_ref[...], preferred_element_type=F32)
                    + emb_ref[...])
        u_ref[:, :H] = eij
        if with_coord:
            sc = _silu(jnp.dot(eij, cmw_ref[...],
                               preferred_element_type=F32)[:, :1]
                       + cmb_ref[...])
            u_ref[:, H:] = sb[:, H:] * sc + k_ref[...]

    ob = blk_off
    return pl.pallas_call(
        body,
        grid=(e // be,),
        in_specs=[
            pl.BlockSpec((be, ROW), lambda i: (i, 0)),
            pl.BlockSpec((d, be), lambda i: (0, i + ob)),
            pl.BlockSpec((d, H), lambda i: (0, 0)),
            pl.BlockSpec((ROW, H), lambda i: (0, 0)),
            pl.BlockSpec((H, H), lambda i: (0, 0)),
            pl.BlockSpec((1, H), lambda i: (0, 0)),
            pl.BlockSpec((H, 8), lambda i: (0, 0)),
            pl.BlockSpec((1, 1), lambda i: (0, 0)),
            pl.BlockSpec((1, H), lambda i: (0, 0)),
        ],
        out_specs=pl.BlockSpec((be, ROW), lambda i: (i, 0)),
        out_shape=jax.ShapeDtypeStruct((e, ROW), F32),
    )(sarr, ea_t, we, mr, emw, emb, cmw8, cmb, konst)


def _tc_node0(x, pos, accs, w1x, w1e, b1, w2, b2, wi1, wj1, bn):
    n, f = x.shape

    def body(x_ref, pos_ref, a0_ref, a1_ref, a2_ref, a3_ref, w1x_ref,
             w1e_ref, b1_ref, w2_ref, b2_ref, wi_ref, wj_ref,
             x1_ref, ti_ref, tj_ref):
        a0 = a0_ref[...] + a1_ref[...] + a2_ref[...] + a3_ref[...]
        e_agg = a0[:, :H]
        csum = a0[:, H:H + 3]
        cnt = a0[:, H + 3:H + 4]
        posn = pos_ref[...] + csum / jnp.maximum(cnt, 1.0)
        h = _silu(jnp.dot(x_ref[...], w1x_ref[...], preferred_element_type=F32)
                  + jnp.dot(e_agg, w1e_ref[...], preferred_element_type=F32)
                  + b1_ref[...])
        x1 = jnp.dot(h, w2_ref[...], preferred_element_type=F32) + b2_ref[...]
        x1_ref[...] = x1
        pad = jnp.zeros((bn, ROW - H - 3), F32)
        ti_ref[...] = jnp.concatenate(
            [jnp.dot(x1, wi_ref[...], preferred_element_type=F32), posn, pad], 1)
        tj_ref[...] = jnp.concatenate(
            [jnp.dot(x1, wj_ref[...], preferred_element_type=F32), -posn, pad], 1)

    c0 = pl.BlockSpec((H, H), lambda i: (0, 0))
    cb = pl.BlockSpec((1, H), lambda i: (0, 0))
    return pl.pallas_call(
        body,
        grid=(n // bn,),
        in_specs=[
            pl.BlockSpec((bn, f), lambda i: (i, 0)),
            pl.BlockSpec((bn, 3), lambda i: (i, 0)),
            pl.BlockSpec((bn, AW), lambda i: (i, 0)),
            pl.BlockSpec((bn, AW), lambda i: (i, 0)),
            pl.BlockSpec((bn, AW), lambda i: (i, 0)),
            pl.BlockSpec((bn, AW), lambda i: (i, 0)),
            pl.BlockSpec((f, H), lambda i: (0, 0)),
            c0, cb, c0, cb, c0, c0,
        ],
        out_specs=[
            pl.BlockSpec((bn, H), lambda i: (i, 0)),
            pl.BlockSpec((bn, ROW), lambda i: (i, 0)),
            pl.BlockSpec((bn, ROW), lambda i: (i, 0)),
        ],
        out_shape=[
            jax.ShapeDtypeStruct((n, H), F32),
            jax.ShapeDtypeStruct((n, ROW), F32),
            jax.ShapeDtypeStruct((n, ROW), F32),
        ],
    )(x, pos, *accs, w1x, w1e, b1, w2, b2, wi1, wj1)


def _tc_node1_pool(x1, accs, batch_col, w1x, w1e, b1, w2, b2, bn):
    n = x1.shape[0]
    g = 64

    def body(x_ref, a0_ref, a1_ref, a2_ref, a3_ref, b_ref, w1x_ref,
             w1e_ref, b1_ref, w2_ref, b2_ref, out_ref):
        e_agg = (a0_ref[:, :H] + a1_ref[:, :H]
                 + a2_ref[:, :H] + a3_ref[:, :H])
        h = _silu(jnp.dot(x_ref[...], w1x_ref[...], preferred_element_type=F32)
                  + jnp.dot(e_agg, w1e_ref[...], preferred_element_type=F32)
                  + b1_ref[...])
        x2 = jnp.dot(h, w2_ref[...], preferred_element_type=F32) + b2_ref[...]
        gid = lax.broadcasted_iota(jnp.int32, (1, g), 1)
        oh = (b_ref[...] == gid).astype(F32)
        feat = jnp.concatenate(
            [x2, jnp.ones((bn, 1), F32), jnp.zeros((bn, 7), F32)], 1)
        contrib = lax.dot_general(oh, feat, (((0,), (0,)), ((), ())),
                                  preferred_element_type=F32)

        @pl.when(pl.program_id(0) == 0)
        def _():
            out_ref[...] = contrib

        @pl.when(pl.program_id(0) != 0)
        def _():
            out_ref[...] = out_ref[...] + contrib

    c0 = pl.BlockSpec((H, H), lambda i: (0, 0))
    cb = pl.BlockSpec((1, H), lambda i: (0, 0))
    return pl.pallas_call(
        body,
        grid=(n // bn,),
        in_specs=[
            pl.BlockSpec((bn, H), lambda i: (i, 0)),
            pl.BlockSpec((bn, AW), lambda i: (i, 0)),
            pl.BlockSpec((bn, AW), lambda i: (i, 0)),
            pl.BlockSpec((bn, AW), lambda i: (i, 0)),
            pl.BlockSpec((bn, AW), lambda i: (i, 0)),
            pl.BlockSpec((bn, 1), lambda i: (i, 0)),
            c0, c0, cb, c0, cb,
        ],
        out_specs=pl.BlockSpec((g, H + 8), lambda i: (0, 0)),
        out_shape=jax.ShapeDtypeStruct((g, H + 8), F32),
    )(x1, *accs, batch_col, w1x, w1e, b1, w2, b2)


def _tc_final(pooled, w1, b1, w2, b2):
    g = pooled.shape[0]
    out_c = w2.shape[1]

    def body(p_ref, w1_ref, b1_ref, w2_ref, b2_ref, o_ref):
        p = p_ref[...]
        xg = p[:, :H] / jnp.maximum(p[:, H:H + 1], 1.0)
        m = jnp.max(xg, axis=1, keepdims=True)
        z = xg - m
        xl = z - jnp.log(jnp.sum(jnp.exp(z), axis=1, keepdims=True))
        hh = jnp.maximum(
            jnp.dot(xl, w1_ref[...], preferred_element_type=F32) + b1_ref[...],
            0.0)
        o_ref[...] = (jnp.dot(hh, w2_ref[...], preferred_element_type=F32)
                      + b2_ref[...])

    return pl.pallas_call(
        body,
        out_shape=jax.ShapeDtypeStruct((g, out_c), F32),
    )(pooled, w1, b1, w2, b2)


def kernel(x, pos, edge_attr, edge_index, batch, params):
    n, f = x.shape
    d = edge_attr.shape[1]
    src = edge_index[0]
    dst = edge_index[1]

    p0, p1, po = params['l0'], params['l1'], params['out']
    w0 = p0['mlp_W']
    wi0, wj0 = w0[:f], w0[f:2 * f]
    we0, wr0 = w0[2 * f:2 * f + d], w0[2 * f + d].reshape(1, H)
    w1 = p1['mlp_W']
    wi1, wj1 = w1[:H], w1[H:2 * H]
    we1, wr1 = w1[2 * H:2 * H + d], w1[2 * H + d].reshape(1, H)

    zrows = jnp.zeros((80, AW), F32)
    batch_col = batch.reshape(n, 1)
    ea_t = edge_attr.T

    ti0, tj0 = _tc_prep_tables(x, pos, wi0, wj0, bn=2000)

    e = src.shape[0]
    eh = e // 2
    bh = eh // 3200

    konst = jnp.zeros((1, H), F32).at[0, 3].set(1.0)

    def run_layer(ti, tj, we, wr, p, with_coord):
        mr = jnp.zeros((ROW, H), F32).at[H:H + 3].set(
            jnp.broadcast_to(wr, (3, H)))
        cmw8 = jnp.concatenate([p['cm_W'], jnp.zeros((H, 7), F32)], 1)
        accs = []
        for ci in range(2):
            sarr = _sc_gather_sum(ti, tj, src, dst, ci * eh, eh)
            u = _tc_edge_mlp(sarr, ea_t, we, mr, p['em_W'],
                             p['em_b'].reshape(1, H), cmw8,
                             p['cm_b'].reshape(1, 1), konst, be=3200,
                             with_coord=with_coord, blk_off=ci * bh)
            a = _sc_scatter_add(u, dst, zrows, ci * eh)
            accs.extend([a[0], a[1]])
        return accs

    accs0 = run_layer(ti0, tj0, we0, wr0, p0, True)
    x1, ti1, tj1 = _tc_node0(
        x, pos, accs0,
        p0['nm_W1'][:f], p0['nm_W1'][f:], p0['nm_b1'].reshape(1, H),
        p0['nm_W2'], p0['nm_b2'].reshape(1, H), wi1, wj1, bn=2000)

    accs1 = run_layer(ti1, tj1, we1, wr1, p1, False)
    pooled = _tc_node1_pool(
        x1, accs1, batch_col,
        p1['nm_W1'][:H], p1['nm_W1'][H:], p1['nm_b1'].reshape(1, H),
        p1['nm_W2'], p1['nm_b2'].reshape(1, H), bn=2000)

    return _tc_final(pooled, po['W1'], po['b1'].reshape(1, H),
                     po['W2'], po['b2'].reshape(1, po['W2'].shape[1]))


# speedup vs baseline: 8.1373x; 1.0024x over previous
import functools

import jax
import jax.numpy as jnp
from jax import lax
from jax.experimental import pallas as pl
from jax.experimental.pallas import tpu as pltpu
from jax.experimental.pallas import tpu_sc as plsc

F32 = jnp.float32
ROW = 128
AW = 72
TWT = 72
H = 64


def _silu(v):
    return v * (1.0 / (1.0 + jnp.exp(-v)))


def _sc_gather_sum(ti, tj, src, dst, eoff, ecnt):
    e = ecnt
    BLK = 64
    nblk = e // BLK
    maxp = (nblk // 32 + 3) // 3
    mesh = plsc.VectorSubcoreMesh(core_axis_name="c", subcore_axis_name="s")

    @functools.partial(
        pl.kernel,
        out_type=jax.ShapeDtypeStruct((e, ROW), F32),
        mesh=mesh,
        scratch_types=[
            [pltpu.VMEM((BLK,), jnp.int32)] * 3,
            [pltpu.VMEM((BLK,), jnp.int32)] * 3,
            [pltpu.VMEM((BLK, TWT), F32)] * 3,
            [pltpu.VMEM((BLK, TWT), F32)] * 3,
            [pltpu.VMEM((BLK, ROW), F32)] * 3,
            [pltpu.SemaphoreType.DMA] * 3,
            [pltpu.SemaphoreType.DMA] * 3,
            [pltpu.SemaphoreType.DMA] * 3,
            [pltpu.SemaphoreType.DMA] * 3,
            [pltpu.SemaphoreType.DMA] * 3,
        ],
        compiler_params=pltpu.CompilerParams(use_tc_tiling_on_sc=False),
    )
    def k(ti_hbm, tj_hbm, src_hbm, dst_hbm, out_hbm,
          idd, ids, ri, rj, ro, sid, sis, sgi, sgj, sw):
        c = lax.axis_index("c")
        s = lax.axis_index("s")
        wid = s * 2 + c
        nb = (nblk - wid + 31) // 32

        def start_idx(q, it):
            @pl.when(it < nb)
            def _():
                e0 = eoff + (wid + it * 32) * BLK
                pltpu.make_async_copy(dst_hbm.at[pl.ds(e0, BLK)], idd[q],
                                      sid[q]).start()
                pltpu.make_async_copy(src_hbm.at[pl.ds(e0, BLK)], ids[q],
                                      sis[q]).start()

        def start_gather(q, it):
            @pl.when(it < nb)
            def _():
                pltpu.make_async_copy(dst_hbm.at[pl.ds(0, BLK)], idd[q],
                                      sid[q]).wait()
                pltpu.make_async_copy(src_hbm.at[pl.ds(0, BLK)], ids[q],
                                      sis[q]).wait()
                pltpu.make_async_copy(ti_hbm.at[idd[q]], ri[q], sgi[q]).start()
                pltpu.make_async_copy(tj_hbm.at[ids[q]], rj[q], sgj[q]).start()

        def finish(q, it):
            @pl.when(it < nb)
            def _():
                e0 = (wid + it * 32) * BLK
                pltpu.make_async_copy(ti_hbm.at[pl.ds(0, BLK)], ri[q],
                                      sgi[q]).wait()
                pltpu.make_async_copy(tj_hbm.at[pl.ds(0, BLK)], rj[q],
                                      sgj[q]).wait()

                @pl.when(it >= 3)
                def _():
                    pltpu.make_async_copy(ro[q], out_hbm.at[pl.ds(0, BLK)],
                                          sw[q]).wait()

                def row(r, carry):
                    for off in (0, 16, 32, 48, TWT - 16):
                        sl = pl.ds(off, 16)
                        ro[q][r, sl] = ri[q][r, sl] + rj[q][r, sl]
                    return carry

                lax.fori_loop(0, BLK, row, 0, unroll=4)
                pltpu.make_async_copy(ro[q], out_hbm.at[pl.ds(e0, BLK)],
                                      sw[q]).start()

        def zrow(r, carry):
            zv = jnp.zeros((16,), F32)
            for q in range(3):
                for off in (72, 88, 104, 112):
                    ro[q][r, pl.ds(off, 16)] = zv
            return carry

        lax.fori_loop(0, BLK, zrow, 0, unroll=4)

        start_idx(0, 0)
        start_idx(1, 1)
        start_idx(2, 2)
        start_gather(0, 0)
        start_gather(1, 1)

        def body(p, carry):
            for q in range(3):
                it = 3 * p + q
                finish(q, it)
                start_idx(q, it + 3)
                start_gather((q + 2) % 3, it + 2)
            return carry

        lax.fori_loop(0, maxp, body, 0)
        for q in range(3):
            pltpu.make_async_copy(ro[q], out_hbm.at[pl.ds(0, BLK)],
                                  sw[q]).wait()

    return k(ti, tj, src, dst)


def _sc_scatter_add(u, dst, zrows, eoff):
    e = u.shape[0]
    n = 10000
    nblk = e // 128
    half = nblk // 2
    maxp = (half // 16 + 2) // 2
    zr = zrows.shape[0]
    mesh = plsc.VectorSubcoreMesh(core_axis_name="c", subcore_axis_name="s")

    @functools.partial(
        pl.kernel,
        out_type=jax.ShapeDtypeStruct((2, n, AW), F32),
        mesh=mesh,
        scratch_types=[
            pltpu.VMEM((128,), jnp.int32), pltpu.VMEM((128,), jnp.int32),
            pltpu.VMEM((128, AW), F32), pltpu.VMEM((128, AW), F32),
            pltpu.VMEM((zr, AW), F32),
            pltpu.VMEM_SHARED((n, AW), F32),
            pltpu.SemaphoreType.DMA, pltpu.SemaphoreType.DMA,
            pltpu.SemaphoreType.DMA, pltpu.SemaphoreType.DMA,
            pltpu.SemaphoreType.DMA, pltpu.SemaphoreType.DMA,
        ],
        compiler_params=pltpu.CompilerParams(use_tc_tiling_on_sc=False),
    )
    def k(u_hbm, dst_hbm, z_hbm, out_hbm, ix0, ix1, u0, u1, z_v, acc,
          li0, lu0, li1, lu1, as0, as1):
        c = lax.axis_index("c")
        s = lax.axis_index("s")
        pltpu.sync_copy(z_hbm, z_v)
        nchunk = jnp.where(s == 15, 5, 8)

        def zbody(kk, carry):
            pltpu.sync_copy(z_v, acc.at[pl.ds(s * 640 + kk * zr, zr)])
            return carry

        lax.fori_loop(0, nchunk, zbody, 0)
        plsc.subcore_barrier()

        nb = (half - s + 15) // 16

        def start(it, ix, uv, li, lu, asem):
            @pl.when(jnp.logical_and(it < nb, it >= 2))
            def _():
                pltpu.make_async_copy(uv, acc.at[pl.ds(0, 128)], asem).wait()

            @pl.when(it < nb)
            def _():
                e0 = (c * half + s + it * 16) * 128
                pltpu.make_async_copy(dst_hbm.at[pl.ds(eoff + e0, 128)], ix,
                                      li).start()
                pltpu.make_async_copy(
                    u_hbm.at[pl.ds(e0, 128), pl.ds(0, AW)], uv, lu).start()

        def process(it, ix, uv, li, lu, asem):
            @pl.when(it < nb)
            def _():
                pltpu.make_async_copy(dst_hbm.at[pl.ds(0, 128)], ix,
                                      li).wait()
                pltpu.make_async_copy(u_hbm.at[pl.ds(0, 128), pl.ds(0, AW)],
                                      uv, lu).wait()
                pltpu.make_async_copy(uv, acc.at[ix], asem).start(add=True)

        start(0, ix0, u0, li0, lu0, as0)
        start(1, ix1, u1, li1, lu1, as1)

        def body(p, carry):
            it = 2 * p
            process(it, ix0, u0, li0, lu0, as0)
            start(it + 2, ix0, u0, li0, lu0, as0)
            process(it + 1, ix1, u1, li1, lu1, as1)
            start(it + 3, ix1, u1, li1, lu1, as1)
            return carry

        lax.fori_loop(0, maxp, body, 0)
        pltpu.make_async_copy(u0, acc.at[pl.ds(0, 128)], as0).wait()
        pltpu.make_async_copy(u1, acc.at[pl.ds(0, 128)], as1).wait()
        plsc.subcore_barrier()

        def wbody(kk, carry):
            r0 = s * 640 + kk * zr
            pltpu.sync_copy(acc.at[pl.ds(r0, zr)], z_v)
            pltpu.sync_copy(z_v, out_hbm.at[c, pl.ds(r0, zr)])
            return carry

        lax.fori_loop(0, nchunk, wbody, 0)

    return k(u, dst, zrows)


def _tc_prep_tables(x, pos, wi, wj, bn):
    n, f = x.shape

    def body(x_ref, pos_ref, wi_ref, wj_ref, ti_ref, tj_ref):
        xb = x_ref[...]
        p = pos_ref[...]
        pad = jnp.zeros((bn, TWT - H - 3), F32)
        ti_ref[...] = jnp.concatenate(
            [jnp.dot(xb, wi_ref[...], preferred_element_type=F32), p, pad], 1)
        tj_ref[...] = jnp.concatenate(
            [jnp.dot(xb, wj_ref[...], preferred_element_type=F32), -p, pad], 1)

    return pl.pallas_call(
        body,
        grid=(n // bn,),
        in_specs=[
            pl.BlockSpec((bn, f), lambda i: (i, 0)),
            pl.BlockSpec((bn, 3), lambda i: (i, 0)),
            pl.BlockSpec((f, H), lambda i: (0, 0)),
            pl.BlockSpec((f, H), lambda i: (0, 0)),
        ],
        out_specs=[
            pl.BlockSpec((bn, TWT), lambda i: (i, 0)),
            pl.BlockSpec((bn, TWT), lambda i: (i, 0)),
        ],
        out_shape=[
            jax.ShapeDtypeStruct((n, TWT), F32),
            jax.ShapeDtypeStruct((n, TWT), F32),
        ],
    )(x, pos, wi, wj)


def _tc_edge_mlp(sarr, ea_t, we, mr, emw, emb, cmw8, cmb, konst, be,
                 with_coord, blk_off=0):
    e = sarr.shape[0]
    d = ea_t.shape[0]

    def body(s_ref, ea_ref, we_ref, mr_ref, emw_ref, emb_ref, cmw_ref,
             cmb_ref, k_ref, u_ref):
        sb = s_ref[...]
        pre = (sb[:, :H]
               + lax.dot_general(ea_ref[...], we_ref[...],
                                 (((0,), (0,)), ((), ())),
                                 preferred_element_type=F32)
               + jnp.dot(sb * sb, mr_ref[...], preferred_element_type=F32))
        er = _silu(pre)
        eij = _silu(jnp.dot(er, emw_ref[...], preferred_element_type=F32)
                    + emb_ref[...])
        u_ref[:, :H] = eij
        if with_coord:
            sc = _silu(jnp.dot(eij, cmw_ref[...],
                               preferred_element_type=F32)[:, :1]
                       + cmb_ref[...])
            u_ref[:, H:] = sb[:, H:] * sc + k_ref[...]

    ob = blk_off
    return pl.pallas_call(
        body,
        grid=(e // be,),
        in_specs=[
            pl.BlockSpec((be, ROW), lambda i: (i, 0)),
            pl.BlockSpec((d, be), lambda i: (0, i + ob)),
            pl.BlockSpec((d, H), lambda i: (0, 0)),
            pl.BlockSpec((ROW, H), lambda i: (0, 0)),
            pl.BlockSpec((H, H), lambda i: (0, 0)),
            pl.BlockSpec((1, H), lambda i: (0, 0)),
            pl.BlockSpec((H, 8), lambda i: (0, 0)),
            pl.BlockSpec((1, 1), lambda i: (0, 0)),
            pl.BlockSpec((1, H), lambda i: (0, 0)),
        ],
        out_specs=pl.BlockSpec((be, ROW), lambda i: (i, 0)),
        out_shape=jax.ShapeDtypeStruct((e, ROW), F32),
    )(sarr, ea_t, we, mr, emw, emb, cmw8, cmb, konst)


def _tc_node0(x, pos, accs, w1x, w1e, b1, w2, b2, wi1, wj1, bn):
    n, f = x.shape

    def body(x_ref, pos_ref, a0_ref, a1_ref, a2_ref, a3_ref, w1x_ref,
             w1e_ref, b1_ref, w2_ref, b2_ref, wi_ref, wj_ref,
             x1_ref, ti_ref, tj_ref):
        a0 = a0_ref[...] + a1_ref[...] + a2_ref[...] + a3_ref[...]
        e_agg = a0[:, :H]
        csum = a0[:, H:H + 3]
        cnt = a0[:, H + 3:H + 4]
        posn = pos_ref[...] + csum / jnp.maximum(cnt, 1.0)
        h = _silu(jnp.dot(x_ref[...], w1x_ref[...], preferred_element_type=F32)
                  + jnp.dot(e_agg, w1e_ref[...], preferred_element_type=F32)
                  + b1_ref[...])
        x1 = jnp.dot(h, w2_ref[...], preferred_element_type=F32) + b2_ref[...]
        x1_ref[...] = x1
        pad = jnp.zeros((bn, TWT - H - 3), F32)
        ti_ref[...] = jnp.concatenate(
            [jnp.dot(x1, wi_ref[...], preferred_element_type=F32), posn, pad], 1)
        tj_ref[...] = jnp.concatenate(
            [jnp.dot(x1, wj_ref[...], preferred_element_type=F32), -posn, pad], 1)

    c0 = pl.BlockSpec((H, H), lambda i: (0, 0))
    cb = pl.BlockSpec((1, H), lambda i: (0, 0))
    return pl.pallas_call(
        body,
        grid=(n // bn,),
        in_specs=[
            pl.BlockSpec((bn, f), lambda i: (i, 0)),
            pl.BlockSpec((bn, 3), lambda i: (i, 0)),
            pl.BlockSpec((bn, AW), lambda i: (i, 0)),
            pl.BlockSpec((bn, AW), lambda i: (i, 0)),
            pl.BlockSpec((bn, AW), lambda i: (i, 0)),
            pl.BlockSpec((bn, AW), lambda i: (i, 0)),
            pl.BlockSpec((f, H), lambda i: (0, 0)),
            c0, cb, c0, cb, c0, c0,
        ],
        out_specs=[
            pl.BlockSpec((bn, H), lambda i: (i, 0)),
            pl.BlockSpec((bn, TWT), lambda i: (i, 0)),
            pl.BlockSpec((bn, TWT), lambda i: (i, 0)),
        ],
        out_shape=[
            jax.ShapeDtypeStruct((n, H), F32),
            jax.ShapeDtypeStruct((n, TWT), F32),
            jax.ShapeDtypeStruct((n, TWT), F32),
        ],
    )(x, pos, *accs, w1x, w1e, b1, w2, b2, wi1, wj1)


def _tc_node1_pool(x1, accs, batch_col, w1x, w1e, b1, w2, b2, bn):
    n = x1.shape[0]
    g = 64

    def body(x_ref, a0_ref, a1_ref, a2_ref, a3_ref, b_ref, w1x_ref,
             w1e_ref, b1_ref, w2_ref, b2_ref, out_ref):
        e_agg = (a0_ref[:, :H] + a1_ref[:, :H]
                 + a2_ref[:, :H] + a3_ref[:, :H])
        h = _silu(jnp.dot(x_ref[...], w1x_ref[...], preferred_element_type=F32)
                  + jnp.dot(e_agg, w1e_ref[...], preferred_element_type=F32)
                  + b1_ref[...])
        x2 = jnp.dot(h, w2_ref[...], preferred_element_type=F32) + b2_ref[...]
        gid = lax.broadcasted_iota(jnp.int32, (1, g), 1)
        oh = (b_ref[...] == gid).astype(F32)
        feat = jnp.concatenate(
            [x2, jnp.ones((bn, 1), F32), jnp.zeros((bn, 7), F32)], 1)
        contrib = lax.dot_general(oh, feat, (((0,), (0,)), ((), ())),
                                  preferred_element_type=F32)

        @pl.when(pl.program_id(0) == 0)
        def _():
            out_ref[...] = contrib

        @pl.when(pl.program_id(0) != 0)
        def _():
            out_ref[...] = out_ref[...] + contrib

    c0 = pl.BlockSpec((H, H), lambda i: (0, 0))
    cb = pl.BlockSpec((1, H), lambda i: (0, 0))
    return pl.pallas_call(
        body,
        grid=(n // bn,),
        in_specs=[
            pl.BlockSpec((bn, H), lambda i: (i, 0)),
            pl.BlockSpec((bn, AW), lambda i: (i, 0)),
            pl.BlockSpec((bn, AW), lambda i: (i, 0)),
            pl.BlockSpec((bn, AW), lambda i: (i, 0)),
            pl.BlockSpec((bn, AW), lambda i: (i, 0)),
            pl.BlockSpec((bn, 1), lambda i: (i, 0)),
            c0, c0, cb, c0, cb,
        ],
        out_specs=pl.BlockSpec((g, H + 8), lambda i: (0, 0)),
        out_shape=jax.ShapeDtypeStruct((g, H + 8), F32),
    )(x1, *accs, batch_col, w1x, w1e, b1, w2, b2)


def _tc_final(pooled, w1, b1, w2, b2):
    g = pooled.shape[0]
    out_c = w2.shape[1]

    def body(p_ref, w1_ref, b1_ref, w2_ref, b2_ref, o_ref):
        p = p_ref[...]
        xg = p[:, :H] / jnp.maximum(p[:, H:H + 1], 1.0)
        m = jnp.max(xg, axis=1, keepdims=True)
        z = xg - m
        xl = z - jnp.log(jnp.sum(jnp.exp(z), axis=1, keepdims=True))
        hh = jnp.maximum(
            jnp.dot(xl, w1_ref[...], preferred_element_type=F32) + b1_ref[...],
            0.0)
        o_ref[...] = (jnp.dot(hh, w2_ref[...], preferred_element_type=F32)
                      + b2_ref[...])

    return pl.pallas_call(
        body,
        out_shape=jax.ShapeDtypeStruct((g, out_c), F32),
    )(pooled, w1, b1, w2, b2)


def kernel(x, pos, edge_attr, edge_index, batch, params):
    n, f = x.shape
    d = edge_attr.shape[1]
    src = edge_index[0]
    dst = edge_index[1]

    p0, p1, po = params['l0'], params['l1'], params['out']
    w0 = p0['mlp_W']
    wi0, wj0 = w0[:f], w0[f:2 * f]
    we0, wr0 = w0[2 * f:2 * f + d], w0[2 * f + d].reshape(1, H)
    w1 = p1['mlp_W']
    wi1, wj1 = w1[:H], w1[H:2 * H]
    we1, wr1 = w1[2 * H:2 * H + d], w1[2 * H + d].reshape(1, H)

    zrows = jnp.zeros((80, AW), F32)
    batch_col = batch.reshape(n, 1)
    ea_t = edge_attr.T

    ti0, tj0 = _tc_prep_tables(x, pos, wi0, wj0, bn=2000)

    e = src.shape[0]
    eh = e // 2
    bh = eh // 3200

    konst = jnp.zeros((1, H), F32).at[0, 3].set(1.0)

    def run_layer(ti, tj, we, wr, p, with_coord):
        mr = jnp.zeros((ROW, H), F32).at[H:H + 3].set(
            jnp.broadcast_to(wr, (3, H)))
        cmw8 = jnp.concatenate([p['cm_W'], jnp.zeros((H, 7), F32)], 1)
        accs = []
        for ci in range(2):
            sarr = _sc_gather_sum(ti, tj, src, dst, ci * eh, eh)
            u = _tc_edge_mlp(sarr, ea_t, we, mr, p['em_W'],
                             p['em_b'].reshape(1, H), cmw8,
                             p['cm_b'].reshape(1, 1), konst, be=3200,
                             with_coord=with_coord, blk_off=ci * bh)
            a = _sc_scatter_add(u, dst, zrows, ci * eh)
            accs.extend([a[0], a[1]])
        return accs

    accs0 = run_layer(ti0, tj0, we0, wr0, p0, True)
    x1, ti1, tj1 = _tc_node0(
        x, pos, accs0,
        p0['nm_W1'][:f], p0['nm_W1'][f:], p0['nm_b1'].reshape(1, H),
        p0['nm_W2'], p0['nm_b2'].reshape(1, H), wi1, wj1, bn=2000)

    accs1 = run_layer(ti1, tj1, we1, wr1, p1, False)
    pooled = _tc_node1_pool(
        x1, accs1, batch_col,
        p1['nm_W1'][:H], p1['nm_W1'][H:], p1['nm_b1'].reshape(1, H),
        p1['nm_W2'], p1['nm_b2'].reshape(1, H), bn=2000)

    return _tc_final(pooled, po['W1'], po['b1'].reshape(1, H),
                     po['W2'], po['b2'].reshape(1, po['W2'].shape[1]))
